# baseline (speedup 1.0000x reference)
import functools

import jax
import jax.numpy as jnp
from jax import lax
from jax.experimental import pallas as pl
from jax.experimental.pallas import tpu as pltpu
from jax.experimental.pallas import tpu_sc as plsc

N = 10000
NS = 2500
NS_PAD = 2560
BC = 128
GRID_C = NS_PAD // BC
KTOT = 32
EPAD = NS_PAD * KTOT
D = 128
TW = 384
EPS = 1e-5
BIG = 2 ** 30


def _fps_body(px_ref, py_ref, pz_ref, src_ref, rows_ref):
    pxv = px_ref[...]
    pyv = py_ref[...]
    pzv = pz_ref[...]
    ii = (lax.broadcasted_iota(jnp.int32, (8, N // 8), 0) * (N // 8)
          + lax.broadcasted_iota(jnp.int32, (8, N // 8), 1))

    def pick(i, j):
        row = src_ref[pl.ds(j, 1)]
        rows_ref[pl.ds(i, 1)] = row
        return row[:, :, 0], row[:, :, 1], row[:, :, 2]

    cx0, cy0, cz0 = pick(0, 0)
    dists0 = jnp.full((8, N // 8), jnp.inf, dtype=jnp.float32)

    def body(i, carry):
        cx, cy, cz, dists = carry
        d = (pxv - cx) * (pxv - cx) + (pyv - cy) * (pyv - cy) \
            + (pzv - cz) * (pzv - cz)
        dists = jnp.minimum(dists, d)
        m = jnp.max(dists)
        j = jnp.min(jnp.where(dists == m, ii, BIG))
        ncx, ncy, ncz = pick(i, j)
        return ncx, ncy, ncz, dists

    lax.fori_loop(1, NS, body, (cx0, cy0, cz0, dists0))


def _fps(px, py, pz, src):
    return pl.pallas_call(
        _fps_body,
        out_shape=jax.ShapeDtypeStruct((NS, 1, 8), jnp.float32),
    )(px, py, pz, src)


def _knn_body(posT_ref, cs_ref, nbr_ref):
    px = posT_ref[0:1, :]
    py = posT_ref[1:2, :]
    pz = posT_ref[2:3, :]
    pv = posT_ref[...]
    cs = cs_ref[...]
    pn = px * px + py * py + pz * pz
    cross = jnp.dot(cs, pv, preferred_element_type=jnp.float32)
    d = pn - (cross + cross)
    ii = lax.broadcasted_iota(jnp.int32, (BC, N), 1)
    ik = lax.broadcasted_iota(jnp.int32, (BC, KTOT), 1)

    def body(k, carry):
        d, nbr = carry
        m = jnp.min(d, axis=1, keepdims=True)
        j = jnp.min(jnp.where(d == m, ii, BIG), axis=1, keepdims=True)
        nbr = jnp.where(ik == k, j, nbr)
        d = jnp.where(ii == j, jnp.inf, d)
        return d, nbr

    _, nbr = lax.fori_loop(0, KTOT, body,
                           (d, jnp.zeros((BC, KTOT), jnp.int32)))
    nbr_ref[...] = nbr


def _knn(posT, cs_pad):
    return pl.pallas_call(
        _knn_body,
        grid=(GRID_C,),
        in_specs=[
            pl.BlockSpec((8, N), lambda i: (0, 0)),
            pl.BlockSpec((BC, 8), lambda i: (i, 0)),
        ],
        out_specs=pl.BlockSpec((BC, KTOT), lambda i: (i, 0)),
        out_shape=jax.ShapeDtypeStruct((NS_PAD, KTOT), jnp.int32),
    )(posT, cs_pad)


def _sc_gather(table, col):
    info = plsc.get_sparse_core_info()
    nw = info.num_cores * info.num_subcores
    rows_per_w = EPAD // nw
    ch = 128
    nch = rows_per_w // ch
    mesh = plsc.VectorSubcoreMesh(core_axis_name="c", subcore_axis_name="s")

    @functools.partial(
        pl.kernel, mesh=mesh,
        out_type=jax.ShapeDtypeStruct((EPAD, TW), jnp.float32),
        scratch_types=[
            pltpu.VMEM((ch,), jnp.int32),
            pltpu.VMEM((ch, TW), jnp.float32),
            pltpu.SemaphoreType.DMA,
        ],
    )
    def gk(table_hbm, col_hbm, out_hbm, idx_v, rows_v, sem):
        wid = lax.axis_index("s") * info.num_cores + lax.axis_index("c")
        base = wid * rows_per_w
        for c in range(nch):
            st = base + c * ch
            pltpu.sync_copy(col_hbm.at[pl.ds(st, ch)], idx_v)
            pltpu.async_copy(table_hbm.at[idx_v], rows_v, sem).wait()
            pltpu.sync_copy(rows_v, out_hbm.at[pl.ds(st, ch)])

    return gk(table, col)


def _table_body(x_ref, p8_ref, w1p0_ref, dwp0_ref, w1p1_ref, dwp1_ref,
                w1x0_ref, dwx0_ref, w1x1_ref, dwx1_ref, t_ref):
    x = x_ref[...]
    p8 = p8_ref[...]

    def mm(a, b):
        return jnp.dot(a, b, preferred_element_type=jnp.float32)

    q0 = mm(x, w1x0_ref[...]) + mm(p8, w1p0_ref[...])
    qd0 = mm(x, dwx0_ref[...]) + mm(p8, dwp0_ref[...])
    q1 = mm(x, w1x1_ref[...]) + mm(p8, w1p1_ref[...])
    qd1 = mm(x, dwx1_ref[...]) + mm(p8, dwp1_ref[...])
    t_ref[...] = jnp.concatenate([q0, qd0, q1, qd1], axis=1)


def _make_table(x, pos8, tws):
    rb = 1000
    full = lambda r, c: pl.BlockSpec((r, c), lambda i: (0, 0))
    in_specs = [pl.BlockSpec((rb, D), lambda i: (i, 0)),
                pl.BlockSpec((rb, 8), lambda i: (i, 0))]
    in_specs += [full(*w.shape) for w in tws]
    return pl.pallas_call(
        _table_body,
        grid=(N // rb,),
        in_specs=in_specs,
        out_specs=pl.BlockSpec((rb, TW), lambda i: (i, 0)),
        out_shape=jax.ShapeDtypeStruct((N, TW), jnp.float32),
    )(x, pos8, *tws)


def _mlpA_body(kk, c1, e_real, o1, od, g_ref, cs_ref, w1p_ref,
               dwp_ref, b1_ref, db_ref, z1_ref, zd_ref, s1_ref, sd_ref):
    i = pl.program_id(0)
    eb = BC * kk
    g = g_ref[...]
    cs = cs_ref[...]

    def rep(a):
        return jnp.broadcast_to(a[:, None, :], (BC, kk, a.shape[1])) \
                  .reshape(eb, a.shape[1])

    def mm(a, b):
        return jnp.dot(a, b, preferred_element_type=jnp.float32)

    z1 = (g[:, :, o1:o1 + c1].reshape(eb, c1)
          - rep(mm(cs, w1p_ref[...])) + b1_ref[...])
    zd = (g[:, :, od:od + D].reshape(eb, D)
          - rep(mm(cs, dwp_ref[...])) + db_ref[...])

    row = lax.broadcasted_iota(jnp.int32, (eb, 1), 0)
    mask = ((row // kk + i * BC) < NS).astype(jnp.float32)

    @pl.when(i == 0)
    def _():
        s1_ref[...] = jnp.zeros_like(s1_ref)
        sd_ref[...] = jnp.zeros_like(sd_ref)

    z1m = z1 * mask
    zdm = zd * mask
    s1_ref[0:1, :] += jnp.sum(z1m, axis=0, keepdims=True)
    s1_ref[1:2, :] += jnp.sum(z1m * z1, axis=0, keepdims=True)
    sd_ref[0:1, :] += jnp.sum(zdm, axis=0, keepdims=True)
    sd_ref[1:2, :] += jnp.sum(zdm * zd, axis=0, keepdims=True)
    z1_ref[...] = z1
    zd_ref[...] = zd


def _bn_coefs(s_ref, e_real, g_ref, bt_ref):
    mu = s_ref[0:1, :] * (1.0 / e_real)
    var = s_ref[1:2, :] * (1.0 / e_real) - mu * mu
    rstd = lax.rsqrt(var + EPS)
    scale = rstd * g_ref[...]
    bias = bt_ref[...] - mu * scale
    return scale, bias


def _mlpB_body(eb, e_real, z_ref, s_ref, g_ref, bt_ref, w_ref, b_ref,
               z2_ref, s2_ref):
    i = pl.program_id(0)
    scale, bias = _bn_coefs(s_ref, e_real, g_ref, bt_ref)
    h = jnp.maximum(z_ref[...] * scale + bias, 0.0)
    z2 = jnp.dot(h, w_ref[...], preferred_element_type=jnp.float32) + b_ref[...]
    row = lax.broadcasted_iota(jnp.int32, (eb, 1), 0)
    mask = ((row + i * eb) < e_real).astype(jnp.float32)

    @pl.when(i == 0)
    def _():
        s2_ref[...] = jnp.zeros_like(s2_ref)

    z2m = z2 * mask
    s2_ref[0:1, :] += jnp.sum(z2m, axis=0, keepdims=True)
    s2_ref[1:2, :] += jnp.sum(z2m * z2, axis=0, keepdims=True)
    z2_ref[...] = z2


def _mlpD_body(kk, e_real, z3_ref, zd_ref, s3_ref, sd_ref, g3_ref, bt3_ref,
               gd_ref, btd_ref, out_ref):
    eb = BC * kk
    sc3, bi3 = _bn_coefs(s3_ref, e_real, g3_ref, bt3_ref)
    scd, bid = _bn_coefs(sd_ref, e_real, gd_ref, btd_ref)
    h = jnp.maximum(z3_ref[...] * sc3 + bi3 + zd_ref[...] * scd + bid, 0.0)
    out_ref[...] = jnp.max(h.reshape(BC, kk, D), axis=1)


def _run_scale(gv, cs_pad, p, kk, o1, od):
    eb = BC * kk
    e_real = NS * kk
    (w1, b1, g1, bt1), (w2, b2, g2, bt2), (w3, b3, g3, bt3) = p["layers"]
    dw, dbl, dg, dbt = p["down"]
    c1, c2, c3 = w1.shape[0], w2.shape[0], w3.shape[0]

    w1pT = jnp.zeros((8, c1), jnp.float32).at[:3].set(w1[:, D:].T)
    dwpT = jnp.zeros((8, D), jnp.float32).at[:3].set(dw[:, D:].T)

    def row(v):
        return v.reshape(1, -1)

    full = lambda r, c: pl.BlockSpec((r, c), lambda i: (0, 0))
    z1, zd, s1, sd = pl.pallas_call(
        functools.partial(_mlpA_body, kk, c1, e_real, o1, od),
        grid=(GRID_C,),
        in_specs=[
            pl.BlockSpec((BC, kk, TW), lambda i: (i, 0, 0)),
            pl.BlockSpec((BC, 8), lambda i: (i, 0)),
            full(8, c1), full(8, D),
            full(1, c1), full(1, D),
        ],
        out_specs=[
            pl.BlockSpec((eb, c1), lambda i: (i, 0)),
            pl.BlockSpec((eb, D), lambda i: (i, 0)),
            full(8, c1), full(8, D),
        ],
        out_shape=[
            jax.ShapeDtypeStruct((GRID_C * eb, c1), jnp.float32),
            jax.ShapeDtypeStruct((GRID_C * eb, D), jnp.float32),
            jax.ShapeDtypeStruct((8, c1), jnp.float32),
            jax.ShapeDtypeStruct((8, D), jnp.float32),
        ],
    )(gv, cs_pad, w1pT, dwpT, row(b1), row(dbl))

    def bc_layer(z, s, g_, bt_, w_, b_, cin, cout):
        return pl.pallas_call(
            functools.partial(_mlpB_body, eb, e_real),
            grid=(GRID_C,),
            in_specs=[
                pl.BlockSpec((eb, cin), lambda i: (i, 0)),
                full(8, cin), full(1, cin), full(1, cin),
                full(cin, cout), full(1, cout),
            ],
            out_specs=[
                pl.BlockSpec((eb, cout), lambda i: (i, 0)),
                full(8, cout),
            ],
            out_shape=[
                jax.ShapeDtypeStruct((GRID_C * eb, cout), jnp.float32),
                jax.ShapeDtypeStruct((8, cout), jnp.float32),
            ],
        )(z, s, row(g_), row(bt_), w_.T, row(b_))

    z2, s2 = bc_layer(z1, s1, g1, bt1, w2, b2, c1, c2)
    z3, s3 = bc_layer(z2, s2, g2, bt2, w3, b3, c2, c3)

    out = pl.pallas_call(
        functools.partial(_mlpD_body, kk, e_real),
        grid=(GRID_C,),
        in_specs=[
            pl.BlockSpec((eb, D), lambda i: (i, 0)),
            pl.BlockSpec((eb, D), lambda i: (i, 0)),
            full(8, D), full(8, D),
            full(1, D), full(1, D), full(1, D), full(1, D),
        ],
        out_specs=pl.BlockSpec((BC, D), lambda i: (i, 0)),
        out_shape=jax.ShapeDtypeStruct((NS_PAD, D), jnp.float32),
    )(z3, zd, s3, sd, row(g3), row(bt3), row(dg), row(dbt))
    return out[:NS]


def kernel(x, pos, batch, params):
    pos = pos.astype(jnp.float32)
    x = x.astype(jnp.float32)
    px = pos[:, 0].reshape(8, N // 8)
    py = pos[:, 1].reshape(8, N // 8)
    pz = pos[:, 2].reshape(8, N // 8)
    src = jnp.zeros((N, 1, 8), jnp.float32)
    src = src.at[:, 0, :3].set(pos).at[:, 0, 3].set(batch.astype(jnp.float32))

    rows = _fps(px, py, pz, src).reshape(NS, 8)
    pos_s = rows[:, :3]
    batch_s = rows[:, 3].astype(jnp.int32)

    cs_pad = jnp.zeros((NS_PAD, 8), jnp.float32).at[:NS].set(rows)
    posT = jnp.zeros((8, N), jnp.float32).at[:3].set(pos.T)
    nbr = _knn(posT, cs_pad)

    pos8 = jnp.zeros((N, 8), jnp.float32).at[:, :3].set(pos)

    def wsplit(w, cout):
        wxT = w[:, :D].T
        wpT = jnp.zeros((8, cout), jnp.float32).at[:3].set(w[:, D:].T)
        return wxT, wpT

    w10, b0 = params[0]["layers"][0][0], params[0]["down"][0]
    w11, b1w = params[1]["layers"][0][0], params[1]["down"][0]
    w1x0, w1p0 = wsplit(w10, 64)
    dwx0, dwp0 = wsplit(b0, D)
    w1x1, w1p1 = wsplit(w11, 64)
    dwx1, dwp1 = wsplit(b1w, D)
    table = _make_table(
        x, pos8, [w1p0, dwp0, w1p1, dwp1, w1x0, dwx0, w1x1, dwx1])

    g = _sc_gather(table, nbr.reshape(-1))
    gv = g.reshape(NS_PAD, KTOT, TW)

    out0 = _run_scale(gv, cs_pad, params[0], 16, 0, 64)
    out1 = _run_scale(gv, cs_pad, params[1], 32, 192, 256)
    return jnp.concatenate([out0, out1], axis=1), pos_s, batch_s


# speedup vs baseline: 1.0195x; 1.0195x over previous
import functools

import jax
import jax.numpy as jnp
from jax import lax
from jax.experimental import pallas as pl
from jax.experimental.pallas import tpu as pltpu
from jax.experimental.pallas import tpu_sc as plsc

N = 10000
NS = 2500
NS_PAD = 2560
BC = 128
GRID_C = NS_PAD // BC
KTOT = 32
EPAD = NS_PAD * KTOT
D = 128
TW = 384
EPS = 1e-5
BIG = 2 ** 30


def _fps_body(px_ref, py_ref, pz_ref, src_ref, rows_ref):
    pxv = px_ref[...]
    pyv = py_ref[...]
    pzv = pz_ref[...]
    ii = (lax.broadcasted_iota(jnp.int32, (8, N // 8), 0) * (N // 8)
          + lax.broadcasted_iota(jnp.int32, (8, N // 8), 1))

    def pick(i, j):
        row = src_ref[pl.ds(j, 1)]
        rows_ref[pl.ds(i, 1)] = row
        return row[:, :, 0], row[:, :, 1], row[:, :, 2]

    cx0, cy0, cz0 = pick(0, 0)
    dists0 = jnp.full((8, N // 8), jnp.inf, dtype=jnp.float32)

    def body(i, carry):
        cx, cy, cz, dists = carry
        d = (pxv - cx) * (pxv - cx) + (pyv - cy) * (pyv - cy) \
            + (pzv - cz) * (pzv - cz)
        dists = jnp.minimum(dists, d)
        m = jnp.max(dists)
        j = jnp.min(jnp.where(dists == m, ii, BIG))
        ncx, ncy, ncz = pick(i, j)
        return ncx, ncy, ncz, dists

    lax.fori_loop(1, NS, body, (cx0, cy0, cz0, dists0))


def _fps(px, py, pz, src):
    return pl.pallas_call(
        _fps_body,
        out_shape=jax.ShapeDtypeStruct((NS, 1, 8), jnp.float32),
    )(px, py, pz, src)


def _knn_body(posT_ref, cs_ref, nbr_ref):
    px = posT_ref[0:1, :]
    py = posT_ref[1:2, :]
    pz = posT_ref[2:3, :]
    cs = cs_ref[...]
    cx = cs[:, 0:1]
    cy = cs[:, 1:2]
    cz = cs[:, 2:3]
    dx = cx - px
    dy = cy - py
    dz = cz - pz
    d = dx * dx + dy * dy + dz * dz
    ii = lax.broadcasted_iota(jnp.int32, (BC, N), 1)
    ik = lax.broadcasted_iota(jnp.int32, (BC, KTOT), 1)

    def body(k, carry):
        d, nbr = carry
        m = jnp.min(d, axis=1, keepdims=True)
        j = jnp.min(jnp.where(d == m, ii, BIG), axis=1, keepdims=True)
        nbr = jnp.where(ik == k, j, nbr)
        d = jnp.where(ii == j, jnp.inf, d)
        return d, nbr

    _, nbr = lax.fori_loop(0, KTOT, body,
                           (d, jnp.zeros((BC, KTOT), jnp.int32)))
    nbr_ref[...] = nbr


def _knn(posT, cs_pad):
    return pl.pallas_call(
        _knn_body,
        grid=(GRID_C,),
        in_specs=[
            pl.BlockSpec((8, N), lambda i: (0, 0)),
            pl.BlockSpec((BC, 8), lambda i: (i, 0)),
        ],
        out_specs=pl.BlockSpec((BC, KTOT), lambda i: (i, 0)),
        out_shape=jax.ShapeDtypeStruct((NS_PAD, KTOT), jnp.int32),
    )(posT, cs_pad)


def _sc_gather(table, col):
    info = plsc.get_sparse_core_info()
    nw = info.num_cores * info.num_subcores
    rows_per_w = EPAD // nw
    ch = 128
    nch = rows_per_w // ch
    mesh = plsc.VectorSubcoreMesh(core_axis_name="c", subcore_axis_name="s")

    @functools.partial(
        pl.kernel, mesh=mesh,
        out_type=jax.ShapeDtypeStruct((EPAD, TW), jnp.float32),
        scratch_types=[
            pltpu.VMEM((ch,), jnp.int32),
            pltpu.VMEM((ch, TW), jnp.float32),
            pltpu.SemaphoreType.DMA,
        ],
    )
    def gk(table_hbm, col_hbm, out_hbm, idx_v, rows_v, sem):
        wid = lax.axis_index("s") * info.num_cores + lax.axis_index("c")
        base = wid * rows_per_w
        for c in range(nch):
            st = base + c * ch
            pltpu.sync_copy(col_hbm.at[pl.ds(st, ch)], idx_v)
            pltpu.async_copy(table_hbm.at[idx_v], rows_v, sem).wait()
            pltpu.sync_copy(rows_v, out_hbm.at[pl.ds(st, ch)])

    return gk(table, col)


def _table_body(x_ref, p8_ref, w1p0_ref, dwp0_ref, w1p1_ref, dwp1_ref,
                w1x0_ref, dwx0_ref, w1x1_ref, dwx1_ref, t_ref):
    x = x_ref[...]
    p8 = p8_ref[...]

    def mm(a, b):
        return jnp.dot(a, b, preferred_element_type=jnp.float32)

    q0 = mm(x, w1x0_ref[...]) + mm(p8, w1p0_ref[...])
    qd0 = mm(x, dwx0_ref[...]) + mm(p8, dwp0_ref[...])
    q1 = mm(x, w1x1_ref[...]) + mm(p8, w1p1_ref[...])
    qd1 = mm(x, dwx1_ref[...]) + mm(p8, dwp1_ref[...])
    t_ref[...] = jnp.concatenate([q0, qd0, q1, qd1], axis=1)


def _make_table(x, pos8, tws):
    rb = 1000
    full = lambda r, c: pl.BlockSpec((r, c), lambda i: (0, 0))
    in_specs = [pl.BlockSpec((rb, D), lambda i: (i, 0)),
                pl.BlockSpec((rb, 8), lambda i: (i, 0))]
    in_specs += [full(*w.shape) for w in tws]
    return pl.pallas_call(
        _table_body,
        grid=(N // rb,),
        in_specs=in_specs,
        out_specs=pl.BlockSpec((rb, TW), lambda i: (i, 0)),
        out_shape=jax.ShapeDtypeStruct((N, TW), jnp.float32),
    )(x, pos8, *tws)


def _mlpA_body(kk, c1, e_real, o1, od, g_ref, cs_ref, w1p_ref,
               dwp_ref, b1_ref, db_ref, z1_ref, zd_ref, s1_ref, sd_ref):
    i = pl.program_id(0)
    eb = BC * kk
    g = g_ref[...]
    cs = cs_ref[...]

    def rep(a):
        return jnp.broadcast_to(a[:, None, :], (BC, kk, a.shape[1])) \
                  .reshape(eb, a.shape[1])

    def mm(a, b):
        return jnp.dot(a, b, preferred_element_type=jnp.float32)

    z1 = (g[:, :, o1:o1 + c1].reshape(eb, c1)
          - rep(mm(cs, w1p_ref[...])) + b1_ref[...])
    zd = (g[:, :, od:od + D].reshape(eb, D)
          - rep(mm(cs, dwp_ref[...])) + db_ref[...])

    row = lax.broadcasted_iota(jnp.int32, (eb, 1), 0)
    mask = ((row // kk + i * BC) < NS).astype(jnp.float32)

    @pl.when(i == 0)
    def _():
        s1_ref[...] = jnp.zeros_like(s1_ref)
        sd_ref[...] = jnp.zeros_like(sd_ref)

    z1m = z1 * mask
    zdm = zd * mask
    s1_ref[0:1, :] += jnp.sum(z1m, axis=0, keepdims=True)
    s1_ref[1:2, :] += jnp.sum(z1m * z1, axis=0, keepdims=True)
    sd_ref[0:1, :] += jnp.sum(zdm, axis=0, keepdims=True)
    sd_ref[1:2, :] += jnp.sum(zdm * zd, axis=0, keepdims=True)
    z1_ref[...] = z1
    zd_ref[...] = zd


def _bn_coefs(s_ref, e_real, g_ref, bt_ref):
    mu = s_ref[0:1, :] * (1.0 / e_real)
    var = s_ref[1:2, :] * (1.0 / e_real) - mu * mu
    rstd = lax.rsqrt(var + EPS)
    scale = rstd * g_ref[...]
    bias = bt_ref[...] - mu * scale
    return scale, bias


def _mlpB_body(eb, e_real, z_ref, s_ref, g_ref, bt_ref, w_ref, b_ref,
               z2_ref, s2_ref):
    i = pl.program_id(0)
    scale, bias = _bn_coefs(s_ref, e_real, g_ref, bt_ref)
    h = jnp.maximum(z_ref[...] * scale + bias, 0.0)
    z2 = jnp.dot(h, w_ref[...], preferred_element_type=jnp.float32) + b_ref[...]
    row = lax.broadcasted_iota(jnp.int32, (eb, 1), 0)
    mask = ((row + i * eb) < e_real).astype(jnp.float32)

    @pl.when(i == 0)
    def _():
        s2_ref[...] = jnp.zeros_like(s2_ref)

    z2m = z2 * mask
    s2_ref[0:1, :] += jnp.sum(z2m, axis=0, keepdims=True)
    s2_ref[1:2, :] += jnp.sum(z2m * z2, axis=0, keepdims=True)
    z2_ref[...] = z2


def _mlpD_body(kk, e_real, z3_ref, zd_ref, s3_ref, sd_ref, g3_ref, bt3_ref,
               gd_ref, btd_ref, out_ref):
    eb = BC * kk
    sc3, bi3 = _bn_coefs(s3_ref, e_real, g3_ref, bt3_ref)
    scd, bid = _bn_coefs(sd_ref, e_real, gd_ref, btd_ref)
    h = jnp.maximum(z3_ref[...] * sc3 + bi3 + zd_ref[...] * scd + bid, 0.0)
    out_ref[...] = jnp.max(h.reshape(BC, kk, D), axis=1)


def _run_scale(gv, cs_pad, p, kk, o1, od):
    eb = BC * kk
    e_real = NS * kk
    (w1, b1, g1, bt1), (w2, b2, g2, bt2), (w3, b3, g3, bt3) = p["layers"]
    dw, dbl, dg, dbt = p["down"]
    c1, c2, c3 = w1.shape[0], w2.shape[0], w3.shape[0]

    w1pT = jnp.zeros((8, c1), jnp.float32).at[:3].set(w1[:, D:].T)
    dwpT = jnp.zeros((8, D), jnp.float32).at[:3].set(dw[:, D:].T)

    def row(v):
        return v.reshape(1, -1)

    full = lambda r, c: pl.BlockSpec((r, c), lambda i: (0, 0))
    z1, zd, s1, sd = pl.pallas_call(
        functools.partial(_mlpA_body, kk, c1, e_real, o1, od),
        grid=(GRID_C,),
        in_specs=[
            pl.BlockSpec((BC, kk, TW), lambda i: (i, 0, 0)),
            pl.BlockSpec((BC, 8), lambda i: (i, 0)),
            full(8, c1), full(8, D),
            full(1, c1), full(1, D),
        ],
        out_specs=[
            pl.BlockSpec((eb, c1), lambda i: (i, 0)),
            pl.BlockSpec((eb, D), lambda i: (i, 0)),
            full(8, c1), full(8, D),
        ],
        out_shape=[
            jax.ShapeDtypeStruct((GRID_C * eb, c1), jnp.float32),
            jax.ShapeDtypeStruct((GRID_C * eb, D), jnp.float32),
            jax.ShapeDtypeStruct((8, c1), jnp.float32),
            jax.ShapeDtypeStruct((8, D), jnp.float32),
        ],
    )(gv, cs_pad, w1pT, dwpT, row(b1), row(dbl))

    def bc_layer(z, s, g_, bt_, w_, b_, cin, cout):
        return pl.pallas_call(
            functools.partial(_mlpB_body, eb, e_real),
            grid=(GRID_C,),
            in_specs=[
                pl.BlockSpec((eb, cin), lambda i: (i, 0)),
                full(8, cin), full(1, cin), full(1, cin),
                full(cin, cout), full(1, cout),
            ],
            out_specs=[
                pl.BlockSpec((eb, cout), lambda i: (i, 0)),
                full(8, cout),
            ],
            out_shape=[
                jax.ShapeDtypeStruct((GRID_C * eb, cout), jnp.float32),
                jax.ShapeDtypeStruct((8, cout), jnp.float32),
            ],
        )(z, s, row(g_), row(bt_), w_.T, row(b_))

    z2, s2 = bc_layer(z1, s1, g1, bt1, w2, b2, c1, c2)
    z3, s3 = bc_layer(z2, s2, g2, bt2, w3, b3, c2, c3)

    out = pl.pallas_call(
        functools.partial(_mlpD_body, kk, e_real),
        grid=(GRID_C,),
        in_specs=[
            pl.BlockSpec((eb, D), lambda i: (i, 0)),
            pl.BlockSpec((eb, D), lambda i: (i, 0)),
            full(8, D), full(8, D),
            full(1, D), full(1, D), full(1, D), full(1, D),
        ],
        out_specs=pl.BlockSpec((BC, D), lambda i: (i, 0)),
        out_shape=jax.ShapeDtypeStruct((NS_PAD, D), jnp.float32),
    )(z3, zd, s3, sd, row(g3), row(bt3), row(dg), row(dbt))
    return out[:NS]


def kernel(x, pos, batch, params):
    pos = pos.astype(jnp.float32)
    x = x.astype(jnp.float32)
    px = pos[:, 0].reshape(8, N // 8)
    py = pos[:, 1].reshape(8, N // 8)
    pz = pos[:, 2].reshape(8, N // 8)
    src = jnp.zeros((N, 1, 8), jnp.float32)
    src = src.at[:, 0, :3].set(pos).at[:, 0, 3].set(batch.astype(jnp.float32))

    rows = _fps(px, py, pz, src).reshape(NS, 8)
    pos_s = rows[:, :3]
    batch_s = rows[:, 3].astype(jnp.int32)

    cs_pad = jnp.zeros((NS_PAD, 8), jnp.float32).at[:NS].set(rows)
    posT = jnp.zeros((8, N), jnp.float32).at[:3].set(pos.T)
    nbr = _knn(posT, cs_pad)

    pos8 = jnp.zeros((N, 8), jnp.float32).at[:, :3].set(pos)

    def wsplit(w, cout):
        wxT = w[:, :D].T
        wpT = jnp.zeros((8, cout), jnp.float32).at[:3].set(w[:, D:].T)
        return wxT, wpT

    w10, b0 = params[0]["layers"][0][0], params[0]["down"][0]
    w11, b1w = params[1]["layers"][0][0], params[1]["down"][0]
    w1x0, w1p0 = wsplit(w10, 64)
    dwx0, dwp0 = wsplit(b0, D)
    w1x1, w1p1 = wsplit(w11, 64)
    dwx1, dwp1 = wsplit(b1w, D)
    table = _make_table(
        x, pos8, [w1p0, dwp0, w1p1, dwp1, w1x0, dwx0, w1x1, dwx1])

    g = _sc_gather(table, nbr.reshape(-1))
    gv = g.reshape(NS_PAD, KTOT, TW)

    out0 = _run_scale(gv, cs_pad, params[0], 16, 0, 64)
    out1 = _run_scale(gv, cs_pad, params[1], 32, 192, 256)
    return jnp.concatenate([out0, out1], axis=1), pos_s, batch_s


# speedup vs baseline: 1.2956x; 1.2709x over previous
import functools

import jax
import jax.numpy as jnp
from jax import lax
from jax.experimental import pallas as pl
from jax.experimental.pallas import tpu as pltpu
from jax.experimental.pallas import tpu_sc as plsc

N = 10000
NS = 2500
NS_PAD = 2560
BC = 128
GRID_C = NS_PAD // BC
KTOT = 32
EPAD = NS_PAD * KTOT
D = 128
TW = 384
EPS = 1e-5
BIG = 2 ** 30


def _fps_body(px_ref, py_ref, pz_ref, pb_ref, rows_ref):
    w = N // 8
    pxv = px_ref[...]
    pyv = py_ref[...]
    pzv = pz_ref[...]
    pbv = pb_ref[...]
    p32 = jnp.concatenate([pxv, pyv, pzv, pbv], axis=0)
    ii = (lax.broadcasted_iota(jnp.int32, (8, w), 0) * w
          + lax.broadcasted_iota(jnp.int32, (8, w), 1))
    ii32 = jnp.concatenate([ii, ii, ii, ii], axis=0)

    def pick(i, j):
        e = jnp.where(ii32 == j, p32, 0.0).reshape(4, 8, w)
        v4 = jnp.sum(jnp.sum(e, axis=2), axis=1, keepdims=True)
        rows_ref[pl.ds(i, 1)] = v4.reshape(1, 4, 1)
        return v4[0:1, :], v4[1:2, :], v4[2:3, :]

    cx0, cy0, cz0 = pick(0, 0)
    dists0 = jnp.full((8, w), jnp.inf, dtype=jnp.float32)

    def body(i, carry):
        cx, cy, cz, dists = carry
        d = (pxv - cx) * (pxv - cx) + (pyv - cy) * (pyv - cy) \
            + (pzv - cz) * (pzv - cz)
        dists = jnp.minimum(dists, d)
        m = jnp.max(dists)
        j = jnp.min(jnp.where(dists == m, ii, BIG))
        ncx, ncy, ncz = pick(i, j)
        return ncx, ncy, ncz, dists

    lax.fori_loop(1, NS, body, (cx0, cy0, cz0, dists0))


def _fps(px, py, pz, pb):
    return pl.pallas_call(
        _fps_body,
        out_shape=jax.ShapeDtypeStruct((NS, 4, 1), jnp.float32),
    )(px, py, pz, pb)


def _knn_body(posT_ref, cs_ref, nbr_ref):
    px = posT_ref[0:1, :]
    py = posT_ref[1:2, :]
    pz = posT_ref[2:3, :]
    cs = cs_ref[...]
    cx = cs[:, 0:1]
    cy = cs[:, 1:2]
    cz = cs[:, 2:3]
    dx = cx - px
    dy = cy - py
    dz = cz - pz
    d = dx * dx + dy * dy + dz * dz
    ii = lax.broadcasted_iota(jnp.int32, (BC, N), 1)
    ik = lax.broadcasted_iota(jnp.int32, (BC, KTOT), 1)

    def body(k, carry):
        d, nbr = carry
        m = jnp.min(d, axis=1, keepdims=True)
        eq = d == m
        j = jnp.min(jnp.where(eq, ii, BIG), axis=1, keepdims=True)
        nbr = jnp.where(ik == k, j, nbr)
        d = jnp.where(eq, jnp.inf, d)
        return d, nbr

    _, nbr = lax.fori_loop(0, KTOT, body,
                           (d, jnp.zeros((BC, KTOT), jnp.int32)))
    nbr_ref[...] = nbr


def _knn(posT, cs_pad):
    return pl.pallas_call(
        _knn_body,
        grid=(GRID_C,),
        in_specs=[
            pl.BlockSpec((8, N), lambda i: (0, 0)),
            pl.BlockSpec((BC, 8), lambda i: (i, 0)),
        ],
        out_specs=pl.BlockSpec((BC, KTOT), lambda i: (i, 0)),
        out_shape=jax.ShapeDtypeStruct((NS_PAD, KTOT), jnp.int32),
    )(posT, cs_pad)


def _sc_gather(table, col):
    info = plsc.get_sparse_core_info()
    nw = info.num_cores * info.num_subcores
    rows_per_w = EPAD // nw
    ch = 128
    nch = rows_per_w // ch
    mesh = plsc.VectorSubcoreMesh(core_axis_name="c", subcore_axis_name="s")

    @functools.partial(
        pl.kernel, mesh=mesh,
        out_type=jax.ShapeDtypeStruct((EPAD, TW), jnp.float32),
        scratch_types=[
            pltpu.VMEM((ch,), jnp.int32),
            pltpu.VMEM((ch, TW), jnp.float32),
            pltpu.SemaphoreType.DMA,
        ],
    )
    def gk(table_hbm, col_hbm, out_hbm, idx_v, rows_v, sem):
        wid = lax.axis_index("s") * info.num_cores + lax.axis_index("c")
        base = wid * rows_per_w
        for c in range(nch):
            st = base + c * ch
            pltpu.sync_copy(col_hbm.at[pl.ds(st, ch)], idx_v)
            pltpu.async_copy(table_hbm.at[idx_v], rows_v, sem).wait()
            pltpu.sync_copy(rows_v, out_hbm.at[pl.ds(st, ch)])

    return gk(table, col)


def _table_body(x_ref, p8_ref, w1p0_ref, dwp0_ref, w1p1_ref, dwp1_ref,
                w1x0_ref, dwx0_ref, w1x1_ref, dwx1_ref, t_ref):
    x = x_ref[...]
    p8 = p8_ref[...]

    def mm(a, b):
        return jnp.dot(a, b, preferred_element_type=jnp.float32)

    q0 = mm(x, w1x0_ref[...]) + mm(p8, w1p0_ref[...])
    qd0 = mm(x, dwx0_ref[...]) + mm(p8, dwp0_ref[...])
    q1 = mm(x, w1x1_ref[...]) + mm(p8, w1p1_ref[...])
    qd1 = mm(x, dwx1_ref[...]) + mm(p8, dwp1_ref[...])
    t_ref[...] = jnp.concatenate([q0, qd0, q1, qd1], axis=1)


def _make_table(x, pos8, tws):
    rb = 1000
    full = lambda r, c: pl.BlockSpec((r, c), lambda i: (0, 0))
    in_specs = [pl.BlockSpec((rb, D), lambda i: (i, 0)),
                pl.BlockSpec((rb, 8), lambda i: (i, 0))]
    in_specs += [full(*w.shape) for w in tws]
    return pl.pallas_call(
        _table_body,
        grid=(N // rb,),
        in_specs=in_specs,
        out_specs=pl.BlockSpec((rb, TW), lambda i: (i, 0)),
        out_shape=jax.ShapeDtypeStruct((N, TW), jnp.float32),
    )(x, pos8, *tws)


def _mlpA_body(kk, c1, e_real, o1, od, g_ref, cs_ref, w1p_ref,
               dwp_ref, b1_ref, db_ref, z1_ref, zd_ref, s1_ref, sd_ref):
    i = pl.program_id(0)
    eb = BC * kk
    g = g_ref[...]
    cs = cs_ref[...]

    def rep(a):
        return jnp.broadcast_to(a[:, None, :], (BC, kk, a.shape[1])) \
                  .reshape(eb, a.shape[1])

    def mm(a, b):
        return jnp.dot(a, b, preferred_element_type=jnp.float32)

    z1 = (g[:, :, o1:o1 + c1].reshape(eb, c1)
          - rep(mm(cs, w1p_ref[...])) + b1_ref[...])
    zd = (g[:, :, od:od + D].reshape(eb, D)
          - rep(mm(cs, dwp_ref[...])) + db_ref[...])

    row = lax.broadcasted_iota(jnp.int32, (eb, 1), 0)
    mask = ((row // kk + i * BC) < NS).astype(jnp.float32)

    @pl.when(i == 0)
    def _():
        s1_ref[...] = jnp.zeros_like(s1_ref)
        sd_ref[...] = jnp.zeros_like(sd_ref)

    z1m = z1 * mask
    zdm = zd * mask
    s1_ref[0:1, :] += jnp.sum(z1m, axis=0, keepdims=True)
    s1_ref[1:2, :] += jnp.sum(z1m * z1, axis=0, keepdims=True)
    sd_ref[0:1, :] += jnp.sum(zdm, axis=0, keepdims=True)
    sd_ref[1:2, :] += jnp.sum(zdm * zd, axis=0, keepdims=True)
    z1_ref[...] = z1
    zd_ref[...] = zd


def _bn_coefs(s_ref, e_real, g_ref, bt_ref):
    mu = s_ref[0:1, :] * (1.0 / e_real)
    var = s_ref[1:2, :] * (1.0 / e_real) - mu * mu
    rstd = lax.rsqrt(var + EPS)
    scale = rstd * g_ref[...]
    bias = bt_ref[...] - mu * scale
    return scale, bias


def _mlpB_body(eb, e_real, z_ref, s_ref, g_ref, bt_ref, w_ref, b_ref,
               z2_ref, s2_ref):
    i = pl.program_id(0)
    scale, bias = _bn_coefs(s_ref, e_real, g_ref, bt_ref)
    h = jnp.maximum(z_ref[...] * scale + bias, 0.0)
    z2 = jnp.dot(h, w_ref[...], preferred_element_type=jnp.float32) + b_ref[...]
    row = lax.broadcasted_iota(jnp.int32, (eb, 1), 0)
    mask = ((row + i * eb) < e_real).astype(jnp.float32)

    @pl.when(i == 0)
    def _():
        s2_ref[...] = jnp.zeros_like(s2_ref)

    z2m = z2 * mask
    s2_ref[0:1, :] += jnp.sum(z2m, axis=0, keepdims=True)
    s2_ref[1:2, :] += jnp.sum(z2m * z2, axis=0, keepdims=True)
    z2_ref[...] = z2


def _mlpD_body(kk, e_real, z3_ref, zd_ref, s3_ref, sd_ref, g3_ref, bt3_ref,
               gd_ref, btd_ref, out_ref):
    eb = BC * kk
    sc3, bi3 = _bn_coefs(s3_ref, e_real, g3_ref, bt3_ref)
    scd, bid = _bn_coefs(sd_ref, e_real, gd_ref, btd_ref)
    h = jnp.maximum(z3_ref[...] * sc3 + bi3 + zd_ref[...] * scd + bid, 0.0)
    out_ref[...] = jnp.max(h.reshape(BC, kk, D), axis=1)


def _run_scale(gv, cs_pad, p, kk, o1, od):
    eb = BC * kk
    e_real = NS * kk
    (w1, b1, g1, bt1), (w2, b2, g2, bt2), (w3, b3, g3, bt3) = p["layers"]
    dw, dbl, dg, dbt = p["down"]
    c1, c2, c3 = w1.shape[0], w2.shape[0], w3.shape[0]

    w1pT = jnp.zeros((8, c1), jnp.float32).at[:3].set(w1[:, D:].T)
    dwpT = jnp.zeros((8, D), jnp.float32).at[:3].set(dw[:, D:].T)

    def row(v):
        return v.reshape(1, -1)

    full = lambda r, c: pl.BlockSpec((r, c), lambda i: (0, 0))
    z1, zd, s1, sd = pl.pallas_call(
        functools.partial(_mlpA_body, kk, c1, e_real, o1, od),
        grid=(GRID_C,),
        in_specs=[
            pl.BlockSpec((BC, kk, TW), lambda i: (i, 0, 0)),
            pl.BlockSpec((BC, 8), lambda i: (i, 0)),
            full(8, c1), full(8, D),
            full(1, c1), full(1, D),
        ],
        out_specs=[
            pl.BlockSpec((eb, c1), lambda i: (i, 0)),
            pl.BlockSpec((eb, D), lambda i: (i, 0)),
            full(8, c1), full(8, D),
        ],
        out_shape=[
            jax.ShapeDtypeStruct((GRID_C * eb, c1), jnp.float32),
            jax.ShapeDtypeStruct((GRID_C * eb, D), jnp.float32),
            jax.ShapeDtypeStruct((8, c1), jnp.float32),
            jax.ShapeDtypeStruct((8, D), jnp.float32),
        ],
    )(gv, cs_pad, w1pT, dwpT, row(b1), row(dbl))

    def bc_layer(z, s, g_, bt_, w_, b_, cin, cout):
        return pl.pallas_call(
            functools.partial(_mlpB_body, eb, e_real),
            grid=(GRID_C,),
            in_specs=[
                pl.BlockSpec((eb, cin), lambda i: (i, 0)),
                full(8, cin), full(1, cin), full(1, cin),
                full(cin, cout), full(1, cout),
            ],
            out_specs=[
                pl.BlockSpec((eb, cout), lambda i: (i, 0)),
                full(8, cout),
            ],
            out_shape=[
                jax.ShapeDtypeStruct((GRID_C * eb, cout), jnp.float32),
                jax.ShapeDtypeStruct((8, cout), jnp.float32),
            ],
        )(z, s, row(g_), row(bt_), w_.T, row(b_))

    z2, s2 = bc_layer(z1, s1, g1, bt1, w2, b2, c1, c2)
    z3, s3 = bc_layer(z2, s2, g2, bt2, w3, b3, c2, c3)

    out = pl.pallas_call(
        functools.partial(_mlpD_body, kk, e_real),
        grid=(GRID_C,),
        in_specs=[
            pl.BlockSpec((eb, D), lambda i: (i, 0)),
            pl.BlockSpec((eb, D), lambda i: (i, 0)),
            full(8, D), full(8, D),
            full(1, D), full(1, D), full(1, D), full(1, D),
        ],
        out_specs=pl.BlockSpec((BC, D), lambda i: (i, 0)),
        out_shape=jax.ShapeDtypeStruct((NS_PAD, D), jnp.float32),
    )(z3, zd, s3, sd, row(g3), row(bt3), row(dg), row(dbt))
    return out[:NS]


def kernel(x, pos, batch, params):
    pos = pos.astype(jnp.float32)
    x = x.astype(jnp.float32)
    px = pos[:, 0].reshape(8, N // 8)
    py = pos[:, 1].reshape(8, N // 8)
    pz = pos[:, 2].reshape(8, N // 8)
    pb = batch.astype(jnp.float32).reshape(8, N // 8)

    rows = _fps(px, py, pz, pb).reshape(NS, 4)
    pos_s = rows[:, :3]
    batch_s = rows[:, 3].astype(jnp.int32)

    cs_pad = jnp.zeros((NS_PAD, 8), jnp.float32).at[:NS, :4].set(rows)
    posT = jnp.zeros((8, N), jnp.float32).at[:3].set(pos.T)
    nbr = _knn(posT, cs_pad)

    pos8 = jnp.zeros((N, 8), jnp.float32).at[:, :3].set(pos)

    def wsplit(w, cout):
        wxT = w[:, :D].T
        wpT = jnp.zeros((8, cout), jnp.float32).at[:3].set(w[:, D:].T)
        return wxT, wpT

    w10, b0 = params[0]["layers"][0][0], params[0]["down"][0]
    w11, b1w = params[1]["layers"][0][0], params[1]["down"][0]
    w1x0, w1p0 = wsplit(w10, 64)
    dwx0, dwp0 = wsplit(b0, D)
    w1x1, w1p1 = wsplit(w11, 64)
    dwx1, dwp1 = wsplit(b1w, D)
    table = _make_table(
        x, pos8, [w1p0, dwp0, w1p1, dwp1, w1x0, dwx0, w1x1, dwx1])

    g = _sc_gather(table, nbr.reshape(-1))
    gv = g.reshape(NS_PAD, KTOT, TW)

    out0 = _run_scale(gv, cs_pad, params[0], 16, 0, 64)
    out1 = _run_scale(gv, cs_pad, params[1], 32, 192, 256)
    return jnp.concatenate([out0, out1], axis=1), pos_s, batch_s


# speedup vs baseline: 1.3851x; 1.0690x over previous
import functools

import jax
import jax.numpy as jnp
from jax import lax
from jax.experimental import pallas as pl
from jax.experimental.pallas import tpu as pltpu
from jax.experimental.pallas import tpu_sc as plsc

N = 10000
NS = 2500
NS_PAD = 2560
BC = 128
GRID_C = NS_PAD // BC
KTOT = 32
EPAD = NS_PAD * KTOT
D = 128
TW = 384
EPS = 1e-5
BIG = 2 ** 30


def _fps_body(px_ref, py_ref, pz_ref, pb_ref, rows_ref):
    w = N // 8
    pxv = px_ref[...]
    pyv = py_ref[...]
    pzv = pz_ref[...]
    pbv = pb_ref[...]
    ii = (lax.broadcasted_iota(jnp.int32, (8, w), 0) * w
          + lax.broadcasted_iota(jnp.int32, (8, w), 1))
    lane8 = lax.broadcasted_iota(jnp.int32, (1, 1, 8), 2)

    def pick(i, j):
        m = ii == j
        cx = jnp.sum(jnp.where(m, pxv, 0.0))
        cy = jnp.sum(jnp.where(m, pyv, 0.0))
        cz = jnp.sum(jnp.where(m, pzv, 0.0))
        cb = jnp.sum(jnp.where(m, pbv, 0.0))
        vals = jnp.where(
            lane8 == 0, cx,
            jnp.where(lane8 == 1, cy,
                      jnp.where(lane8 == 2, cz,
                                jnp.where(lane8 == 3, cb, 0.0))))
        rows_ref[pl.ds(i, 1)] = vals
        return cx, cy, cz

    cx0, cy0, cz0 = pick(0, 0)
    dists0 = jnp.full((8, w), jnp.inf, dtype=jnp.float32)

    def body(i, carry):
        cx, cy, cz, dists = carry
        d = (pxv - cx) * (pxv - cx) + (pyv - cy) * (pyv - cy) \
            + (pzv - cz) * (pzv - cz)
        dists = jnp.minimum(dists, d)
        m = jnp.max(dists)
        j = jnp.min(jnp.where(dists == m, ii, BIG))
        ncx, ncy, ncz = pick(i, j)
        return ncx, ncy, ncz, dists

    lax.fori_loop(1, NS, body, (cx0, cy0, cz0, dists0))


def _fps(px, py, pz, pb):
    return pl.pallas_call(
        _fps_body,
        out_shape=jax.ShapeDtypeStruct((NS, 1, 8), jnp.float32),
    )(px, py, pz, pb)


FOLD = 8
WSLOT = N // FOLD
INFKEY = 0x7F000000
_SORT8 = [(0, 1), (2, 3), (4, 5), (6, 7), (0, 2), (1, 3), (4, 6), (5, 7),
          (1, 2), (5, 6), (0, 4), (1, 5), (2, 6), (3, 7), (2, 4), (3, 5),
          (1, 2), (3, 4), (5, 6)]


def _knn_body(px_ref, py_ref, pz_ref, cs_ref, nbr_ref):
    px = px_ref[...].reshape(1, FOLD, WSLOT)
    py = py_ref[...].reshape(1, FOLD, WSLOT)
    pz = pz_ref[...].reshape(1, FOLD, WSLOT)
    cs = cs_ref[...]
    cx = cs[:, 0:1].reshape(BC, 1, 1)
    cy = cs[:, 1:2].reshape(BC, 1, 1)
    cz = cs[:, 2:3].reshape(BC, 1, 1)
    dx = cx - px
    dy = cy - py
    dz = cz - pz
    d = dx * dx + dy * dy + dz * dz

    db = lax.bitcast_convert_type(d, jnp.int32)
    sub = lax.broadcasted_iota(jnp.int32, (BC, FOLD, WSLOT), 1)
    keys = (db & ~(FOLD - 1)) | sub

    r = [keys[:, s, :] for s in range(FOLD)]
    for a, b in _SORT8:
        lo = jnp.minimum(r[a], r[b])
        hi = jnp.maximum(r[a], r[b])
        r[a], r[b] = lo, hi

    ii = lax.broadcasted_iota(jnp.int32, (BC, WSLOT), 1)
    ik = lax.broadcasted_iota(jnp.int32, (BC, KTOT), 1)

    def body(k, carry):
        nbr = carry[0]
        r = list(carry[1:])
        m = jnp.min(r[0], axis=1, keepdims=True)
        eq = r[0] == m
        jslot = jnp.min(jnp.where(eq, ii, BIG), axis=1, keepdims=True)
        orig = (m & (FOLD - 1)) * WSLOT + jslot
        nbr = jnp.where(ik == k, orig, nbr)
        hit = ii == jslot
        for s in range(FOLD - 1):
            r[s] = jnp.where(hit, r[s + 1], r[s])
        r[FOLD - 1] = jnp.where(hit, INFKEY, r[FOLD - 1])
        return (nbr, *r)

    out = lax.fori_loop(0, KTOT, body,
                        (jnp.zeros((BC, KTOT), jnp.int32), *r))
    nbr_ref[...] = out[0]


def _knn(px, py, pz, cs_pad):
    return pl.pallas_call(
        _knn_body,
        grid=(GRID_C,),
        in_specs=[
            pl.BlockSpec((8, N // 8), lambda i: (0, 0)),
            pl.BlockSpec((8, N // 8), lambda i: (0, 0)),
            pl.BlockSpec((8, N // 8), lambda i: (0, 0)),
            pl.BlockSpec((BC, 8), lambda i: (i, 0)),
        ],
        out_specs=pl.BlockSpec((BC, KTOT), lambda i: (i, 0)),
        out_shape=jax.ShapeDtypeStruct((NS_PAD, KTOT), jnp.int32),
    )(px, py, pz, cs_pad)


def _sc_gather(table, col):
    info = plsc.get_sparse_core_info()
    nw = info.num_cores * info.num_subcores
    rows_per_w = EPAD // nw
    ch = 128
    nch = rows_per_w // ch
    mesh = plsc.VectorSubcoreMesh(core_axis_name="c", subcore_axis_name="s")

    @functools.partial(
        pl.kernel, mesh=mesh,
        out_type=jax.ShapeDtypeStruct((EPAD, TW), jnp.float32),
        scratch_types=[
            pltpu.VMEM((ch,), jnp.int32),
            pltpu.VMEM((ch, TW), jnp.float32),
            pltpu.SemaphoreType.DMA,
        ],
    )
    def gk(table_hbm, col_hbm, out_hbm, idx_v, rows_v, sem):
        wid = lax.axis_index("s") * info.num_cores + lax.axis_index("c")
        base = wid * rows_per_w
        for c in range(nch):
            st = base + c * ch
            pltpu.sync_copy(col_hbm.at[pl.ds(st, ch)], idx_v)
            pltpu.async_copy(table_hbm.at[idx_v], rows_v, sem).wait()
            pltpu.sync_copy(rows_v, out_hbm.at[pl.ds(st, ch)])

    return gk(table, col)


def _table_body(x_ref, p8_ref, w1p0_ref, dwp0_ref, w1p1_ref, dwp1_ref,
                w1x0_ref, dwx0_ref, w1x1_ref, dwx1_ref, t_ref):
    x = x_ref[...]
    p8 = p8_ref[...]

    def mm(a, b):
        return jnp.dot(a, b, preferred_element_type=jnp.float32)

    q0 = mm(x, w1x0_ref[...]) + mm(p8, w1p0_ref[...])
    qd0 = mm(x, dwx0_ref[...]) + mm(p8, dwp0_ref[...])
    q1 = mm(x, w1x1_ref[...]) + mm(p8, w1p1_ref[...])
    qd1 = mm(x, dwx1_ref[...]) + mm(p8, dwp1_ref[...])
    t_ref[...] = jnp.concatenate([q0, qd0, q1, qd1], axis=1)


def _make_table(x, pos8, tws):
    rb = 1000
    full = lambda r, c: pl.BlockSpec((r, c), lambda i: (0, 0))
    in_specs = [pl.BlockSpec((rb, D), lambda i: (i, 0)),
                pl.BlockSpec((rb, 8), lambda i: (i, 0))]
    in_specs += [full(*w.shape) for w in tws]
    return pl.pallas_call(
        _table_body,
        grid=(N // rb,),
        in_specs=in_specs,
        out_specs=pl.BlockSpec((rb, TW), lambda i: (i, 0)),
        out_shape=jax.ShapeDtypeStruct((N, TW), jnp.float32),
    )(x, pos8, *tws)


def _mlpA_body(kk, c1, e_real, o1, od, g_ref, cs_ref, w1p_ref,
               dwp_ref, b1_ref, db_ref, z1_ref, zd_ref, s1_ref, sd_ref):
    i = pl.program_id(0)
    eb = BC * kk
    g = g_ref[...]
    cs = cs_ref[...]

    def rep(a):
        return jnp.broadcast_to(a[:, None, :], (BC, kk, a.shape[1])) \
                  .reshape(eb, a.shape[1])

    def mm(a, b):
        return jnp.dot(a, b, preferred_element_type=jnp.float32)

    z1 = (g[:, :, o1:o1 + c1].reshape(eb, c1)
          - rep(mm(cs, w1p_ref[...])) + b1_ref[...])
    zd = (g[:, :, od:od + D].reshape(eb, D)
          - rep(mm(cs, dwp_ref[...])) + db_ref[...])

    row = lax.broadcasted_iota(jnp.int32, (eb, 1), 0)
    mask = ((row // kk + i * BC) < NS).astype(jnp.float32)

    @pl.when(i == 0)
    def _():
        s1_ref[...] = jnp.zeros_like(s1_ref)
        sd_ref[...] = jnp.zeros_like(sd_ref)

    z1m = z1 * mask
    zdm = zd * mask
    s1_ref[0:1, :] += jnp.sum(z1m, axis=0, keepdims=True)
    s1_ref[1:2, :] += jnp.sum(z1m * z1, axis=0, keepdims=True)
    sd_ref[0:1, :] += jnp.sum(zdm, axis=0, keepdims=True)
    sd_ref[1:2, :] += jnp.sum(zdm * zd, axis=0, keepdims=True)
    z1_ref[...] = z1
    zd_ref[...] = zd


def _bn_coefs(s_ref, e_real, g_ref, bt_ref):
    mu = s_ref[0:1, :] * (1.0 / e_real)
    var = s_ref[1:2, :] * (1.0 / e_real) - mu * mu
    rstd = lax.rsqrt(var + EPS)
    scale = rstd * g_ref[...]
    bias = bt_ref[...] - mu * scale
    return scale, bias


def _mlpB_body(eb, e_real, z_ref, s_ref, g_ref, bt_ref, w_ref, b_ref,
               z2_ref, s2_ref):
    i = pl.program_id(0)
    scale, bias = _bn_coefs(s_ref, e_real, g_ref, bt_ref)
    h = jnp.maximum(z_ref[...] * scale + bias, 0.0)
    z2 = jnp.dot(h, w_ref[...], preferred_element_type=jnp.float32) + b_ref[...]
    row = lax.broadcasted_iota(jnp.int32, (eb, 1), 0)
    mask = ((row + i * eb) < e_real).astype(jnp.float32)

    @pl.when(i == 0)
    def _():
        s2_ref[...] = jnp.zeros_like(s2_ref)

    z2m = z2 * mask
    s2_ref[0:1, :] += jnp.sum(z2m, axis=0, keepdims=True)
    s2_ref[1:2, :] += jnp.sum(z2m * z2, axis=0, keepdims=True)
    z2_ref[...] = z2


def _mlpD_body(kk, e_real, z3_ref, zd_ref, s3_ref, sd_ref, g3_ref, bt3_ref,
               gd_ref, btd_ref, out_ref):
    eb = BC * kk
    sc3, bi3 = _bn_coefs(s3_ref, e_real, g3_ref, bt3_ref)
    scd, bid = _bn_coefs(sd_ref, e_real, gd_ref, btd_ref)
    h = jnp.maximum(z3_ref[...] * sc3 + bi3 + zd_ref[...] * scd + bid, 0.0)
    out_ref[...] = jnp.max(h.reshape(BC, kk, D), axis=1)


def _run_scale(gv, cs_pad, p, kk, o1, od):
    eb = BC * kk
    e_real = NS * kk
    (w1, b1, g1, bt1), (w2, b2, g2, bt2), (w3, b3, g3, bt3) = p["layers"]
    dw, dbl, dg, dbt = p["down"]
    c1, c2, c3 = w1.shape[0], w2.shape[0], w3.shape[0]

    w1pT = jnp.zeros((8, c1), jnp.float32).at[:3].set(w1[:, D:].T)
    dwpT = jnp.zeros((8, D), jnp.float32).at[:3].set(dw[:, D:].T)

    def row(v):
        return v.reshape(1, -1)

    full = lambda r, c: pl.BlockSpec((r, c), lambda i: (0, 0))
    z1, zd, s1, sd = pl.pallas_call(
        functools.partial(_mlpA_body, kk, c1, e_real, o1, od),
        grid=(GRID_C,),
        in_specs=[
            pl.BlockSpec((BC, kk, TW), lambda i: (i, 0, 0)),
            pl.BlockSpec((BC, 8), lambda i: (i, 0)),
            full(8, c1), full(8, D),
            full(1, c1), full(1, D),
        ],
        out_specs=[
            pl.BlockSpec((eb, c1), lambda i: (i, 0)),
            pl.BlockSpec((eb, D), lambda i: (i, 0)),
            full(8, c1), full(8, D),
        ],
        out_shape=[
            jax.ShapeDtypeStruct((GRID_C * eb, c1), jnp.float32),
            jax.ShapeDtypeStruct((GRID_C * eb, D), jnp.float32),
            jax.ShapeDtypeStruct((8, c1), jnp.float32),
            jax.ShapeDtypeStruct((8, D), jnp.float32),
        ],
    )(gv, cs_pad, w1pT, dwpT, row(b1), row(dbl))

    def bc_layer(z, s, g_, bt_, w_, b_, cin, cout):
        return pl.pallas_call(
            functools.partial(_mlpB_body, eb, e_real),
            grid=(GRID_C,),
            in_specs=[
                pl.BlockSpec((eb, cin), lambda i: (i, 0)),
                full(8, cin), full(1, cin), full(1, cin),
                full(cin, cout), full(1, cout),
            ],
            out_specs=[
                pl.BlockSpec((eb, cout), lambda i: (i, 0)),
                full(8, cout),
            ],
            out_shape=[
                jax.ShapeDtypeStruct((GRID_C * eb, cout), jnp.float32),
                jax.ShapeDtypeStruct((8, cout), jnp.float32),
            ],
        )(z, s, row(g_), row(bt_), w_.T, row(b_))

    z2, s2 = bc_layer(z1, s1, g1, bt1, w2, b2, c1, c2)
    z3, s3 = bc_layer(z2, s2, g2, bt2, w3, b3, c2, c3)

    out = pl.pallas_call(
        functools.partial(_mlpD_body, kk, e_real),
        grid=(GRID_C,),
        in_specs=[
            pl.BlockSpec((eb, D), lambda i: (i, 0)),
            pl.BlockSpec((eb, D), lambda i: (i, 0)),
            full(8, D), full(8, D),
            full(1, D), full(1, D), full(1, D), full(1, D),
        ],
        out_specs=pl.BlockSpec((BC, D), lambda i: (i, 0)),
        out_shape=jax.ShapeDtypeStruct((NS_PAD, D), jnp.float32),
    )(z3, zd, s3, sd, row(g3), row(bt3), row(dg), row(dbt))
    return out[:NS]


def kernel(x, pos, batch, params):
    pos = pos.astype(jnp.float32)
    x = x.astype(jnp.float32)
    px = pos[:, 0].reshape(8, N // 8)
    py = pos[:, 1].reshape(8, N // 8)
    pz = pos[:, 2].reshape(8, N // 8)
    pb = batch.astype(jnp.float32).reshape(8, N // 8)

    rows = _fps(px, py, pz, pb).reshape(NS, 8)
    pos_s = rows[:, :3]
    batch_s = rows[:, 3].astype(jnp.int32)

    cs_pad = jnp.zeros((NS_PAD, 8), jnp.float32).at[:NS].set(rows)
    nbr = _knn(px, py, pz, cs_pad)

    pos8 = jnp.zeros((N, 8), jnp.float32).at[:, :3].set(pos)

    def wsplit(w, cout):
        wxT = w[:, :D].T
        wpT = jnp.zeros((8, cout), jnp.float32).at[:3].set(w[:, D:].T)
        return wxT, wpT

    w10, b0 = params[0]["layers"][0][0], params[0]["down"][0]
    w11, b1w = params[1]["layers"][0][0], params[1]["down"][0]
    w1x0, w1p0 = wsplit(w10, 64)
    dwx0, dwp0 = wsplit(b0, D)
    w1x1, w1p1 = wsplit(w11, 64)
    dwx1, dwp1 = wsplit(b1w, D)
    table = _make_table(
        x, pos8, [w1p0, dwp0, w1p1, dwp1, w1x0, dwx0, w1x1, dwx1])

    g = _sc_gather(table, nbr.reshape(-1))
    gv = g.reshape(NS_PAD, KTOT, TW)

    out0 = _run_scale(gv, cs_pad, params[0], 16, 0, 64)
    out1 = _run_scale(gv, cs_pad, params[1], 32, 192, 256)
    return jnp.concatenate([out0, out1], axis=1), pos_s, batch_s


# speedup vs baseline: 1.5180x; 1.0960x over previous
import functools

import jax
import jax.numpy as jnp
from jax import lax
from jax.experimental import pallas as pl
from jax.experimental.pallas import tpu as pltpu
from jax.experimental.pallas import tpu_sc as plsc

N = 10000
NS = 2500
NS_PAD = 2560
BC = 128
GRID_C = NS_PAD // BC
KTOT = 32
EPAD = NS_PAD * KTOT
D = 128
TW = 384
EPS = 1e-5
BIG = 2 ** 30


def _fps_body(px_ref, py_ref, pz_ref, pb_ref, rows_ref):
    w = N // 8
    pxv = px_ref[...]
    pyv = py_ref[...]
    pzv = pz_ref[...]
    pbv = pb_ref[...]
    ii = (lax.broadcasted_iota(jnp.int32, (8, w), 0) * w
          + lax.broadcasted_iota(jnp.int32, (8, w), 1))
    lane8 = lax.broadcasted_iota(jnp.int32, (1, 1, 8), 2)

    def pick(i, j):
        m = ii == j
        cx = jnp.sum(jnp.where(m, pxv, 0.0))
        cy = jnp.sum(jnp.where(m, pyv, 0.0))
        cz = jnp.sum(jnp.where(m, pzv, 0.0))
        cb = jnp.sum(jnp.where(m, pbv, 0.0))
        vals = jnp.where(
            lane8 == 0, cx,
            jnp.where(lane8 == 1, cy,
                      jnp.where(lane8 == 2, cz,
                                jnp.where(lane8 == 3, cb, 0.0))))
        rows_ref[pl.ds(i, 1)] = vals
        return cx, cy, cz

    cx0, cy0, cz0 = pick(0, 0)
    dists0 = jnp.full((8, w), jnp.inf, dtype=jnp.float32)

    def body(i, carry):
        cx, cy, cz, dists = carry
        d = (pxv - cx) * (pxv - cx) + (pyv - cy) * (pyv - cy) \
            + (pzv - cz) * (pzv - cz)
        dists = jnp.minimum(dists, d)
        m = jnp.max(dists)
        j = jnp.min(jnp.where(dists == m, ii, BIG))
        ncx, ncy, ncz = pick(i, j)
        return ncx, ncy, ncz, dists

    lax.fori_loop(1, NS, body, (cx0, cy0, cz0, dists0), unroll=2)


def _fps(px, py, pz, pb):
    return pl.pallas_call(
        _fps_body,
        out_shape=jax.ShapeDtypeStruct((NS, 1, 8), jnp.float32),
    )(px, py, pz, pb)


FOLD = 8
WSLOT = N // FOLD
INFKEY = 0x7F000000
_SORT8 = [(0, 1), (2, 3), (4, 5), (6, 7), (0, 2), (1, 3), (4, 6), (5, 7),
          (1, 2), (5, 6), (0, 4), (1, 5), (2, 6), (3, 7), (2, 4), (3, 5),
          (1, 2), (3, 4), (5, 6)]


BCK = 128


def _knn_body(px_ref, py_ref, pz_ref, cs_ref, nbr_ref):
    px = px_ref[...].reshape(1, FOLD, WSLOT)
    py = py_ref[...].reshape(1, FOLD, WSLOT)
    pz = pz_ref[...].reshape(1, FOLD, WSLOT)
    cs = cs_ref[...]
    cx = cs[:, 0:1].reshape(BCK, 1, 1)
    cy = cs[:, 1:2].reshape(BCK, 1, 1)
    cz = cs[:, 2:3].reshape(BCK, 1, 1)
    dx = cx - px
    dy = cy - py
    dz = cz - pz
    d = dx * dx + dy * dy + dz * dz

    db = lax.bitcast_convert_type(d, jnp.int32)
    sub = lax.broadcasted_iota(jnp.int32, (BCK, FOLD, WSLOT), 1)
    keys = (db & ~(FOLD - 1)) | sub

    r = [keys[:, s, :] for s in range(FOLD)]
    for a, b in _SORT8:
        lo = jnp.minimum(r[a], r[b])
        hi = jnp.maximum(r[a], r[b])
        r[a], r[b] = lo, hi

    ii = lax.broadcasted_iota(jnp.int32, (BCK, WSLOT), 1)
    ik = lax.broadcasted_iota(jnp.int32, (BCK, KTOT), 1)

    def body(k, carry):
        nbr = carry[0]
        r = list(carry[1:])
        m = jnp.min(r[0], axis=1, keepdims=True)
        eq = r[0] == m
        jslot = jnp.min(jnp.where(eq, ii, BIG), axis=1, keepdims=True)
        orig = (m & (FOLD - 1)) * WSLOT + jslot
        nbr = jnp.where(ik == k, orig, nbr)
        hit = ii == jslot
        for s in range(FOLD - 1):
            r[s] = jnp.where(hit, r[s + 1], r[s])
        r[FOLD - 1] = jnp.where(hit, INFKEY, r[FOLD - 1])
        return (nbr, *r)

    out = lax.fori_loop(0, KTOT, body,
                        (jnp.zeros((BCK, KTOT), jnp.int32), *r), unroll=2)
    nbr_ref[...] = out[0]


def _knn(px, py, pz, cs_pad):
    return pl.pallas_call(
        _knn_body,
        grid=(NS_PAD // BCK,),
        in_specs=[
            pl.BlockSpec((8, N // 8), lambda i: (0, 0)),
            pl.BlockSpec((8, N // 8), lambda i: (0, 0)),
            pl.BlockSpec((8, N // 8), lambda i: (0, 0)),
            pl.BlockSpec((BCK, 8), lambda i: (i, 0)),
        ],
        out_specs=pl.BlockSpec((BCK, KTOT), lambda i: (i, 0)),
        out_shape=jax.ShapeDtypeStruct((NS_PAD, KTOT), jnp.int32),
    )(px, py, pz, cs_pad)


def _sc_gather(table, col):
    info = plsc.get_sparse_core_info()
    nw = info.num_cores * info.num_subcores
    rows_per_w = EPAD // nw
    ch = 128
    nch = rows_per_w // ch
    mesh = plsc.VectorSubcoreMesh(core_axis_name="c", subcore_axis_name="s")

    @functools.partial(
        pl.kernel, mesh=mesh,
        out_type=jax.ShapeDtypeStruct((EPAD, TW), jnp.float32),
        scratch_types=[
            pltpu.VMEM((ch,), jnp.int32),
            pltpu.VMEM((ch, TW), jnp.float32),
            pltpu.SemaphoreType.DMA,
        ],
    )
    def gk(table_hbm, col_hbm, out_hbm, idx_v, rows_v, sem):
        wid = lax.axis_index("s") * info.num_cores + lax.axis_index("c")
        base = wid * rows_per_w
        for c in range(nch):
            st = base + c * ch
            pltpu.sync_copy(col_hbm.at[pl.ds(st, ch)], idx_v)
            pltpu.async_copy(table_hbm.at[idx_v], rows_v, sem).wait()
            pltpu.sync_copy(rows_v, out_hbm.at[pl.ds(st, ch)])

    return gk(table, col)


def _table_body(x_ref, p8_ref, w1p0_ref, dwp0_ref, w1p1_ref, dwp1_ref,
                w1x0_ref, dwx0_ref, w1x1_ref, dwx1_ref, t_ref):
    x = x_ref[...]
    p8 = p8_ref[...]

    def mm(a, b):
        return jnp.dot(a, b, preferred_element_type=jnp.float32)

    q0 = mm(x, w1x0_ref[...]) + mm(p8, w1p0_ref[...])
    qd0 = mm(x, dwx0_ref[...]) + mm(p8, dwp0_ref[...])
    q1 = mm(x, w1x1_ref[...]) + mm(p8, w1p1_ref[...])
    qd1 = mm(x, dwx1_ref[...]) + mm(p8, dwp1_ref[...])
    t_ref[...] = jnp.concatenate([q0, qd0, q1, qd1], axis=1)


def _make_table(x, pos8, tws):
    rb = 1000
    full = lambda r, c: pl.BlockSpec((r, c), lambda i: (0, 0))
    in_specs = [pl.BlockSpec((rb, D), lambda i: (i, 0)),
                pl.BlockSpec((rb, 8), lambda i: (i, 0))]
    in_specs += [full(*w.shape) for w in tws]
    return pl.pallas_call(
        _table_body,
        grid=(N // rb,),
        in_specs=in_specs,
        out_specs=pl.BlockSpec((rb, TW), lambda i: (i, 0)),
        out_shape=jax.ShapeDtypeStruct((N, TW), jnp.float32),
    )(x, pos8, *tws)


def _mlpA_body(kk, c1, e_real, o1, od, g_ref, cs_ref, w1p_ref,
               dwp_ref, b1_ref, db_ref, z1_ref, zd_ref, s1_ref, sd_ref):
    i = pl.program_id(0)
    eb = BC * kk
    g = g_ref[...]
    cs = cs_ref[...]

    def rep(a):
        return jnp.broadcast_to(a[:, None, :], (BC, kk, a.shape[1])) \
                  .reshape(eb, a.shape[1])

    def mm(a, b):
        return jnp.dot(a, b, preferred_element_type=jnp.float32)

    z1 = (g[:, :, o1:o1 + c1].reshape(eb, c1)
          - rep(mm(cs, w1p_ref[...])) + b1_ref[...])
    zd = (g[:, :, od:od + D].reshape(eb, D)
          - rep(mm(cs, dwp_ref[...])) + db_ref[...])

    row = lax.broadcasted_iota(jnp.int32, (eb, 1), 0)
    mask = ((row // kk + i * BC) < NS).astype(jnp.float32)

    @pl.when(i == 0)
    def _():
        s1_ref[...] = jnp.zeros_like(s1_ref)
        sd_ref[...] = jnp.zeros_like(sd_ref)

    z1m = z1 * mask
    zdm = zd * mask
    s1_ref[0:1, :] += jnp.sum(z1m, axis=0, keepdims=True)
    s1_ref[1:2, :] += jnp.sum(z1m * z1, axis=0, keepdims=True)
    sd_ref[0:1, :] += jnp.sum(zdm, axis=0, keepdims=True)
    sd_ref[1:2, :] += jnp.sum(zdm * zd, axis=0, keepdims=True)
    z1_ref[...] = z1
    zd_ref[...] = zd


def _bn_coefs(s_ref, e_real, g_ref, bt_ref):
    mu = s_ref[0:1, :] * (1.0 / e_real)
    var = s_ref[1:2, :] * (1.0 / e_real) - mu * mu
    rstd = lax.rsqrt(var + EPS)
    scale = rstd * g_ref[...]
    bias = bt_ref[...] - mu * scale
    return scale, bias


def _mlpB_body(eb, e_real, z_ref, s_ref, g_ref, bt_ref, w_ref, b_ref,
               z2_ref, s2_ref):
    i = pl.program_id(0)
    scale, bias = _bn_coefs(s_ref, e_real, g_ref, bt_ref)
    h = jnp.maximum(z_ref[...] * scale + bias, 0.0)
    z2 = jnp.dot(h, w_ref[...], preferred_element_type=jnp.float32) + b_ref[...]
    row = lax.broadcasted_iota(jnp.int32, (eb, 1), 0)
    mask = ((row + i * eb) < e_real).astype(jnp.float32)

    @pl.when(i == 0)
    def _():
        s2_ref[...] = jnp.zeros_like(s2_ref)

    z2m = z2 * mask
    s2_ref[0:1, :] += jnp.sum(z2m, axis=0, keepdims=True)
    s2_ref[1:2, :] += jnp.sum(z2m * z2, axis=0, keepdims=True)
    z2_ref[...] = z2


def _mlpD_body(kk, e_real, z3_ref, zd_ref, s3_ref, sd_ref, g3_ref, bt3_ref,
               gd_ref, btd_ref, out_ref):
    eb = BC * kk
    sc3, bi3 = _bn_coefs(s3_ref, e_real, g3_ref, bt3_ref)
    scd, bid = _bn_coefs(sd_ref, e_real, gd_ref, btd_ref)
    h = jnp.maximum(z3_ref[...] * sc3 + bi3 + zd_ref[...] * scd + bid, 0.0)
    out_ref[...] = jnp.max(h.reshape(BC, kk, D), axis=1)


def _run_scale(gv, cs_pad, p, kk, o1, od):
    eb = BC * kk
    e_real = NS * kk
    (w1, b1, g1, bt1), (w2, b2, g2, bt2), (w3, b3, g3, bt3) = p["layers"]
    dw, dbl, dg, dbt = p["down"]
    c1, c2, c3 = w1.shape[0], w2.shape[0], w3.shape[0]

    w1pT = jnp.zeros((8, c1), jnp.float32).at[:3].set(w1[:, D:].T)
    dwpT = jnp.zeros((8, D), jnp.float32).at[:3].set(dw[:, D:].T)

    def row(v):
        return v.reshape(1, -1)

    full = lambda r, c: pl.BlockSpec((r, c), lambda i: (0, 0))
    z1, zd, s1, sd = pl.pallas_call(
        functools.partial(_mlpA_body, kk, c1, e_real, o1, od),
        grid=(GRID_C,),
        in_specs=[
            pl.BlockSpec((BC, kk, TW), lambda i: (i, 0, 0)),
            pl.BlockSpec((BC, 8), lambda i: (i, 0)),
            full(8, c1), full(8, D),
            full(1, c1), full(1, D),
        ],
        out_specs=[
            pl.BlockSpec((eb, c1), lambda i: (i, 0)),
            pl.BlockSpec((eb, D), lambda i: (i, 0)),
            full(8, c1), full(8, D),
        ],
        out_shape=[
            jax.ShapeDtypeStruct((GRID_C * eb, c1), jnp.float32),
            jax.ShapeDtypeStruct((GRID_C * eb, D), jnp.float32),
            jax.ShapeDtypeStruct((8, c1), jnp.float32),
            jax.ShapeDtypeStruct((8, D), jnp.float32),
        ],
    )(gv, cs_pad, w1pT, dwpT, row(b1), row(dbl))

    def bc_layer(z, s, g_, bt_, w_, b_, cin, cout):
        return pl.pallas_call(
            functools.partial(_mlpB_body, eb, e_real),
            grid=(GRID_C,),
            in_specs=[
                pl.BlockSpec((eb, cin), lambda i: (i, 0)),
                full(8, cin), full(1, cin), full(1, cin),
                full(cin, cout), full(1, cout),
            ],
            out_specs=[
                pl.BlockSpec((eb, cout), lambda i: (i, 0)),
                full(8, cout),
            ],
            out_shape=[
                jax.ShapeDtypeStruct((GRID_C * eb, cout), jnp.float32),
                jax.ShapeDtypeStruct((8, cout), jnp.float32),
            ],
        )(z, s, row(g_), row(bt_), w_.T, row(b_))

    z2, s2 = bc_layer(z1, s1, g1, bt1, w2, b2, c1, c2)
    z3, s3 = bc_layer(z2, s2, g2, bt2, w3, b3, c2, c3)

    out = pl.pallas_call(
        functools.partial(_mlpD_body, kk, e_real),
        grid=(GRID_C,),
        in_specs=[
            pl.BlockSpec((eb, D), lambda i: (i, 0)),
            pl.BlockSpec((eb, D), lambda i: (i, 0)),
            full(8, D), full(8, D),
            full(1, D), full(1, D), full(1, D), full(1, D),
        ],
        out_specs=pl.BlockSpec((BC, D), lambda i: (i, 0)),
        out_shape=jax.ShapeDtypeStruct((NS_PAD, D), jnp.float32),
    )(z3, zd, s3, sd, row(g3), row(bt3), row(dg), row(dbt))
    return out[:NS]


def kernel(x, pos, batch, params):
    pos = pos.astype(jnp.float32)
    x = x.astype(jnp.float32)
    px = pos[:, 0].reshape(8, N // 8)
    py = pos[:, 1].reshape(8, N // 8)
    pz = pos[:, 2].reshape(8, N // 8)
    pb = batch.astype(jnp.float32).reshape(8, N // 8)

    rows = _fps(px, py, pz, pb).reshape(NS, 8)
    pos_s = rows[:, :3]
    batch_s = rows[:, 3].astype(jnp.int32)

    cs_pad = jnp.zeros((NS_PAD, 8), jnp.float32).at[:NS].set(rows)
    nbr = _knn(px, py, pz, cs_pad)

    pos8 = jnp.zeros((N, 8), jnp.float32).at[:, :3].set(pos)

    def wsplit(w, cout):
        wxT = w[:, :D].T
        wpT = jnp.zeros((8, cout), jnp.float32).at[:3].set(w[:, D:].T)
        return wxT, wpT

    w10, b0 = params[0]["layers"][0][0], params[0]["down"][0]
    w11, b1w = params[1]["layers"][0][0], params[1]["down"][0]
    w1x0, w1p0 = wsplit(w10, 64)
    dwx0, dwp0 = wsplit(b0, D)
    w1x1, w1p1 = wsplit(w11, 64)
    dwx1, dwp1 = wsplit(b1w, D)
    table = _make_table(
        x, pos8, [w1p0, dwp0, w1p1, dwp1, w1x0, dwx0, w1x1, dwx1])

    g = _sc_gather(table, nbr.reshape(-1))
    gv = g.reshape(NS_PAD, KTOT, TW)

    out0 = _run_scale(gv, cs_pad, params[0], 16, 0, 64)
    out1 = _run_scale(gv, cs_pad, params[1], 32, 192, 256)
    return jnp.concatenate([out0, out1], axis=1), pos_s, batch_s


# speedup vs baseline: 1.5923x; 1.0489x over previous
import functools

import jax
import jax.numpy as jnp
from jax import lax
from jax.experimental import pallas as pl
from jax.experimental.pallas import tpu as pltpu
from jax.experimental.pallas import tpu_sc as plsc

N = 10000
NS = 2500
NS_PAD = 2560
BC = 128
GRID_C = NS_PAD // BC
KTOT = 32
EPAD = NS_PAD * KTOT
D = 128
TW = 384
EPS = 1e-5
BIG = 2 ** 30


def _fps_body(px_ref, py_ref, pz_ref, pb_ref, rows_ref):
    w = N // 8
    pxv = px_ref[...]
    pyv = py_ref[...]
    pzv = pz_ref[...]
    pbv = pb_ref[...]
    ii = (lax.broadcasted_iota(jnp.int32, (8, w), 0) * w
          + lax.broadcasted_iota(jnp.int32, (8, w), 1))
    lane8 = lax.broadcasted_iota(jnp.int32, (1, 1, 8), 2)

    def pick(i, j):
        m = ii == j
        cx = jnp.sum(jnp.where(m, pxv, 0.0))
        cy = jnp.sum(jnp.where(m, pyv, 0.0))
        cz = jnp.sum(jnp.where(m, pzv, 0.0))
        cb = jnp.sum(jnp.where(m, pbv, 0.0))
        vals = jnp.where(
            lane8 == 0, cx,
            jnp.where(lane8 == 1, cy,
                      jnp.where(lane8 == 2, cz,
                                jnp.where(lane8 == 3, cb, 0.0))))
        rows_ref[pl.ds(i, 1)] = vals
        return cx, cy, cz

    cx0, cy0, cz0 = pick(0, 0)
    dists0 = jnp.full((8, w), jnp.inf, dtype=jnp.float32)

    def body(i, carry):
        cx, cy, cz, dists = carry
        d = (pxv - cx) * (pxv - cx) + (pyv - cy) * (pyv - cy) \
            + (pzv - cz) * (pzv - cz)
        dists = jnp.minimum(dists, d)
        m = jnp.max(dists)
        j = jnp.min(jnp.where(dists == m, ii, BIG))
        ncx, ncy, ncz = pick(i, j)
        return ncx, ncy, ncz, dists

    lax.fori_loop(1, NS, body, (cx0, cy0, cz0, dists0), unroll=4)


def _fps(px, py, pz, pb):
    return pl.pallas_call(
        _fps_body,
        out_shape=jax.ShapeDtypeStruct((NS, 1, 8), jnp.float32),
    )(px, py, pz, pb)


FOLD = 8
WSLOT = N // FOLD
INFKEY = 0x7F000000
_SORT8 = [(0, 1), (2, 3), (4, 5), (6, 7), (0, 2), (1, 3), (4, 6), (5, 7),
          (1, 2), (5, 6), (0, 4), (1, 5), (2, 6), (3, 7), (2, 4), (3, 5),
          (1, 2), (3, 4), (5, 6)]


BCK = 128


def _knn_body(px_ref, py_ref, pz_ref, cs_ref, nbr_ref):
    px = px_ref[...].reshape(1, FOLD, WSLOT)
    py = py_ref[...].reshape(1, FOLD, WSLOT)
    pz = pz_ref[...].reshape(1, FOLD, WSLOT)
    cs = cs_ref[...]
    cx = cs[:, 0:1].reshape(BCK, 1, 1)
    cy = cs[:, 1:2].reshape(BCK, 1, 1)
    cz = cs[:, 2:3].reshape(BCK, 1, 1)
    dx = cx - px
    dy = cy - py
    dz = cz - pz
    d = dx * dx + dy * dy + dz * dz

    db = lax.bitcast_convert_type(d, jnp.int32)
    sub = lax.broadcasted_iota(jnp.int32, (BCK, FOLD, WSLOT), 1)
    keys = (db & ~(FOLD - 1)) | sub

    r = [keys[:, s, :] for s in range(FOLD)]
    for a, b in _SORT8:
        lo = jnp.minimum(r[a], r[b])
        hi = jnp.maximum(r[a], r[b])
        r[a], r[b] = lo, hi

    ii = lax.broadcasted_iota(jnp.int32, (BCK, WSLOT), 1)
    ik = lax.broadcasted_iota(jnp.int32, (BCK, KTOT), 1)

    def body(k, carry):
        nbr = carry[0]
        r = list(carry[1:])
        m = jnp.min(r[0], axis=1, keepdims=True)
        eq = r[0] == m
        jslot = jnp.min(jnp.where(eq, ii, BIG), axis=1, keepdims=True)
        orig = (m & (FOLD - 1)) * WSLOT + jslot
        nbr = jnp.where(ik == k, orig, nbr)
        hit = ii == jslot
        for s in range(FOLD - 1):
            r[s] = jnp.where(hit, r[s + 1], r[s])
        r[FOLD - 1] = jnp.where(hit, INFKEY, r[FOLD - 1])
        return (nbr, *r)

    out = lax.fori_loop(0, KTOT, body,
                        (jnp.zeros((BCK, KTOT), jnp.int32), *r), unroll=4)
    nbr_ref[...] = out[0]


def _knn(px, py, pz, cs_pad):
    return pl.pallas_call(
        _knn_body,
        grid=(NS_PAD // BCK,),
        in_specs=[
            pl.BlockSpec((8, N // 8), lambda i: (0, 0)),
            pl.BlockSpec((8, N // 8), lambda i: (0, 0)),
            pl.BlockSpec((8, N // 8), lambda i: (0, 0)),
            pl.BlockSpec((BCK, 8), lambda i: (i, 0)),
        ],
        out_specs=pl.BlockSpec((BCK, KTOT), lambda i: (i, 0)),
        out_shape=jax.ShapeDtypeStruct((NS_PAD, KTOT), jnp.int32),
    )(px, py, pz, cs_pad)


def _sc_gather(table, col):
    info = plsc.get_sparse_core_info()
    nw = info.num_cores * info.num_subcores
    rows_per_w = EPAD // nw
    ch = 128
    nch = rows_per_w // ch
    mesh = plsc.VectorSubcoreMesh(core_axis_name="c", subcore_axis_name="s")

    @functools.partial(
        pl.kernel, mesh=mesh,
        out_type=jax.ShapeDtypeStruct((EPAD, TW), jnp.float32),
        scratch_types=[
            pltpu.VMEM((ch,), jnp.int32),
            pltpu.VMEM((ch, TW), jnp.float32),
            pltpu.SemaphoreType.DMA,
        ],
    )
    def gk(table_hbm, col_hbm, out_hbm, idx_v, rows_v, sem):
        wid = lax.axis_index("s") * info.num_cores + lax.axis_index("c")
        base = wid * rows_per_w
        for c in range(nch):
            st = base + c * ch
            pltpu.sync_copy(col_hbm.at[pl.ds(st, ch)], idx_v)
            pltpu.async_copy(table_hbm.at[idx_v], rows_v, sem).wait()
            pltpu.sync_copy(rows_v, out_hbm.at[pl.ds(st, ch)])

    return gk(table, col)


def _table_body(x_ref, p8_ref, w1p0_ref, dwp0_ref, w1p1_ref, dwp1_ref,
                w1x0_ref, dwx0_ref, w1x1_ref, dwx1_ref, t_ref):
    x = x_ref[...]
    p8 = p8_ref[...]

    def mm(a, b):
        return jnp.dot(a, b, preferred_element_type=jnp.float32)

    q0 = mm(x, w1x0_ref[...]) + mm(p8, w1p0_ref[...])
    qd0 = mm(x, dwx0_ref[...]) + mm(p8, dwp0_ref[...])
    q1 = mm(x, w1x1_ref[...]) + mm(p8, w1p1_ref[...])
    qd1 = mm(x, dwx1_ref[...]) + mm(p8, dwp1_ref[...])
    t_ref[...] = jnp.concatenate([q0, qd0, q1, qd1], axis=1)


def _make_table(x, pos8, tws):
    rb = 1000
    full = lambda r, c: pl.BlockSpec((r, c), lambda i: (0, 0))
    in_specs = [pl.BlockSpec((rb, D), lambda i: (i, 0)),
                pl.BlockSpec((rb, 8), lambda i: (i, 0))]
    in_specs += [full(*w.shape) for w in tws]
    return pl.pallas_call(
        _table_body,
        grid=(N // rb,),
        in_specs=in_specs,
        out_specs=pl.BlockSpec((rb, TW), lambda i: (i, 0)),
        out_shape=jax.ShapeDtypeStruct((N, TW), jnp.float32),
    )(x, pos8, *tws)


def _mlpA_body(kk, c1, e_real, o1, od, g_ref, cs_ref, w1p_ref,
               dwp_ref, b1_ref, db_ref, z1_ref, zd_ref, s1_ref, sd_ref):
    i = pl.program_id(0)
    eb = BC * kk
    g = g_ref[...]
    cs = cs_ref[...]

    def rep(a):
        return jnp.broadcast_to(a[:, None, :], (BC, kk, a.shape[1])) \
                  .reshape(eb, a.shape[1])

    def mm(a, b):
        return jnp.dot(a, b, preferred_element_type=jnp.float32)

    z1 = (g[:, :, o1:o1 + c1].reshape(eb, c1)
          - rep(mm(cs, w1p_ref[...])) + b1_ref[...])
    zd = (g[:, :, od:od + D].reshape(eb, D)
          - rep(mm(cs, dwp_ref[...])) + db_ref[...])

    row = lax.broadcasted_iota(jnp.int32, (eb, 1), 0)
    mask = ((row // kk + i * BC) < NS).astype(jnp.float32)

    @pl.when(i == 0)
    def _():
        s1_ref[...] = jnp.zeros_like(s1_ref)
        sd_ref[...] = jnp.zeros_like(sd_ref)

    z1m = z1 * mask
    zdm = zd * mask
    s1_ref[0:1, :] += jnp.sum(z1m, axis=0, keepdims=True)
    s1_ref[1:2, :] += jnp.sum(z1m * z1, axis=0, keepdims=True)
    sd_ref[0:1, :] += jnp.sum(zdm, axis=0, keepdims=True)
    sd_ref[1:2, :] += jnp.sum(zdm * zd, axis=0, keepdims=True)
    z1_ref[...] = z1
    zd_ref[...] = zd


def _bn_coefs(s_ref, e_real, g_ref, bt_ref):
    mu = s_ref[0:1, :] * (1.0 / e_real)
    var = s_ref[1:2, :] * (1.0 / e_real) - mu * mu
    rstd = lax.rsqrt(var + EPS)
    scale = rstd * g_ref[...]
    bias = bt_ref[...] - mu * scale
    return scale, bias


def _mlpB_body(eb, e_real, z_ref, s_ref, g_ref, bt_ref, w_ref, b_ref,
               z2_ref, s2_ref):
    i = pl.program_id(0)
    scale, bias = _bn_coefs(s_ref, e_real, g_ref, bt_ref)
    h = jnp.maximum(z_ref[...] * scale + bias, 0.0)
    z2 = jnp.dot(h, w_ref[...], preferred_element_type=jnp.float32) + b_ref[...]
    row = lax.broadcasted_iota(jnp.int32, (eb, 1), 0)
    mask = ((row + i * eb) < e_real).astype(jnp.float32)

    @pl.when(i == 0)
    def _():
        s2_ref[...] = jnp.zeros_like(s2_ref)

    z2m = z2 * mask
    s2_ref[0:1, :] += jnp.sum(z2m, axis=0, keepdims=True)
    s2_ref[1:2, :] += jnp.sum(z2m * z2, axis=0, keepdims=True)
    z2_ref[...] = z2


def _mlpD_body(kk, e_real, z3_ref, zd_ref, s3_ref, sd_ref, g3_ref, bt3_ref,
               gd_ref, btd_ref, out_ref):
    eb = BC * kk
    sc3, bi3 = _bn_coefs(s3_ref, e_real, g3_ref, bt3_ref)
    scd, bid = _bn_coefs(sd_ref, e_real, gd_ref, btd_ref)
    h = jnp.maximum(z3_ref[...] * sc3 + bi3 + zd_ref[...] * scd + bid, 0.0)
    out_ref[...] = jnp.max(h.reshape(BC, kk, D), axis=1)


def _run_scale(gv, cs_pad, p, kk, o1, od):
    eb = BC * kk
    e_real = NS * kk
    (w1, b1, g1, bt1), (w2, b2, g2, bt2), (w3, b3, g3, bt3) = p["layers"]
    dw, dbl, dg, dbt = p["down"]
    c1, c2, c3 = w1.shape[0], w2.shape[0], w3.shape[0]

    w1pT = jnp.zeros((8, c1), jnp.float32).at[:3].set(w1[:, D:].T)
    dwpT = jnp.zeros((8, D), jnp.float32).at[:3].set(dw[:, D:].T)

    def row(v):
        return v.reshape(1, -1)

    full = lambda r, c: pl.BlockSpec((r, c), lambda i: (0, 0))
    z1, zd, s1, sd = pl.pallas_call(
        functools.partial(_mlpA_body, kk, c1, e_real, o1, od),
        grid=(GRID_C,),
        in_specs=[
            pl.BlockSpec((BC, kk, TW), lambda i: (i, 0, 0)),
            pl.BlockSpec((BC, 8), lambda i: (i, 0)),
            full(8, c1), full(8, D),
            full(1, c1), full(1, D),
        ],
        out_specs=[
            pl.BlockSpec((eb, c1), lambda i: (i, 0)),
            pl.BlockSpec((eb, D), lambda i: (i, 0)),
            full(8, c1), full(8, D),
        ],
        out_shape=[
            jax.ShapeDtypeStruct((GRID_C * eb, c1), jnp.float32),
            jax.ShapeDtypeStruct((GRID_C * eb, D), jnp.float32),
            jax.ShapeDtypeStruct((8, c1), jnp.float32),
            jax.ShapeDtypeStruct((8, D), jnp.float32),
        ],
    )(gv, cs_pad, w1pT, dwpT, row(b1), row(dbl))

    def bc_layer(z, s, g_, bt_, w_, b_, cin, cout):
        return pl.pallas_call(
            functools.partial(_mlpB_body, eb, e_real),
            grid=(GRID_C,),
            in_specs=[
                pl.BlockSpec((eb, cin), lambda i: (i, 0)),
                full(8, cin), full(1, cin), full(1, cin),
                full(cin, cout), full(1, cout),
            ],
            out_specs=[
                pl.BlockSpec((eb, cout), lambda i: (i, 0)),
                full(8, cout),
            ],
            out_shape=[
                jax.ShapeDtypeStruct((GRID_C * eb, cout), jnp.float32),
                jax.ShapeDtypeStruct((8, cout), jnp.float32),
            ],
        )(z, s, row(g_), row(bt_), w_.T, row(b_))

    z2, s2 = bc_layer(z1, s1, g1, bt1, w2, b2, c1, c2)
    z3, s3 = bc_layer(z2, s2, g2, bt2, w3, b3, c2, c3)

    out = pl.pallas_call(
        functools.partial(_mlpD_body, kk, e_real),
        grid=(GRID_C,),
        in_specs=[
            pl.BlockSpec((eb, D), lambda i: (i, 0)),
            pl.BlockSpec((eb, D), lambda i: (i, 0)),
            full(8, D), full(8, D),
            full(1, D), full(1, D), full(1, D), full(1, D),
        ],
        out_specs=pl.BlockSpec((BC, D), lambda i: (i, 0)),
        out_shape=jax.ShapeDtypeStruct((NS_PAD, D), jnp.float32),
    )(z3, zd, s3, sd, row(g3), row(bt3), row(dg), row(dbt))
    return out[:NS]


def kernel(x, pos, batch, params):
    pos = pos.astype(jnp.float32)
    x = x.astype(jnp.float32)
    px = pos[:, 0].reshape(8, N // 8)
    py = pos[:, 1].reshape(8, N // 8)
    pz = pos[:, 2].reshape(8, N // 8)
    pb = batch.astype(jnp.float32).reshape(8, N // 8)

    rows = _fps(px, py, pz, pb).reshape(NS, 8)
    pos_s = rows[:, :3]
    batch_s = rows[:, 3].astype(jnp.int32)

    cs_pad = jnp.zeros((NS_PAD, 8), jnp.float32).at[:NS].set(rows)
    nbr = _knn(px, py, pz, cs_pad)

    pos8 = jnp.zeros((N, 8), jnp.float32).at[:, :3].set(pos)

    def wsplit(w, cout):
        wxT = w[:, :D].T
        wpT = jnp.zeros((8, cout), jnp.float32).at[:3].set(w[:, D:].T)
        return wxT, wpT

    w10, b0 = params[0]["layers"][0][0], params[0]["down"][0]
    w11, b1w = params[1]["layers"][0][0], params[1]["down"][0]
    w1x0, w1p0 = wsplit(w10, 64)
    dwx0, dwp0 = wsplit(b0, D)
    w1x1, w1p1 = wsplit(w11, 64)
    dwx1, dwp1 = wsplit(b1w, D)
    table = _make_table(
        x, pos8, [w1p0, dwp0, w1p1, dwp1, w1x0, dwx0, w1x1, dwx1])

    g = _sc_gather(table, nbr.reshape(-1))
    gv = g.reshape(NS_PAD, KTOT, TW)

    out0 = _run_scale(gv, cs_pad, params[0], 16, 0, 64)
    out1 = _run_scale(gv, cs_pad, params[1], 32, 192, 256)
    return jnp.concatenate([out0, out1], axis=1), pos_s, batch_s


# speedup vs baseline: 1.6204x; 1.0177x over previous
import functools

import jax
import jax.numpy as jnp
from jax import lax
from jax.experimental import pallas as pl
from jax.experimental.pallas import tpu as pltpu
from jax.experimental.pallas import tpu_sc as plsc

N = 10000
NS = 2500
NS_PAD = 2560
BC = 128
GRID_C = NS_PAD // BC
KTOT = 32
EPAD = NS_PAD * KTOT
D = 128
TW = 384
EPS = 1e-5
BIG = 2 ** 30


def _fps_body(px_ref, py_ref, pz_ref, pb_ref, rows_ref):
    w = N // 8
    pxv = px_ref[...]
    pyv = py_ref[...]
    pzv = pz_ref[...]
    pbv = pb_ref[...]
    ii = (lax.broadcasted_iota(jnp.int32, (8, w), 0) * w
          + lax.broadcasted_iota(jnp.int32, (8, w), 1))
    lane8 = lax.broadcasted_iota(jnp.int32, (1, 1, 8), 2)

    def pick(i, j):
        m = ii == j
        cx = jnp.sum(jnp.where(m, pxv, 0.0))
        cy = jnp.sum(jnp.where(m, pyv, 0.0))
        cz = jnp.sum(jnp.where(m, pzv, 0.0))
        cb = jnp.sum(jnp.where(m, pbv, 0.0))
        vals = jnp.where(
            lane8 == 0, cx,
            jnp.where(lane8 == 1, cy,
                      jnp.where(lane8 == 2, cz,
                                jnp.where(lane8 == 3, cb, 0.0))))
        rows_ref[pl.ds(i, 1)] = vals
        return cx, cy, cz

    cx0, cy0, cz0 = pick(0, 0)
    dists0 = jnp.full((8, w), jnp.inf, dtype=jnp.float32)

    def body(i, carry):
        cx, cy, cz, dists = carry
        d = (pxv - cx) * (pxv - cx) + (pyv - cy) * (pyv - cy) \
            + (pzv - cz) * (pzv - cz)
        dists = jnp.minimum(dists, d)
        m = jnp.max(dists)
        j = jnp.min(jnp.where(dists == m, ii, BIG))
        ncx, ncy, ncz = pick(i, j)
        return ncx, ncy, ncz, dists

    lax.fori_loop(1, NS, body, (cx0, cy0, cz0, dists0), unroll=8)


def _fps(px, py, pz, pb):
    return pl.pallas_call(
        _fps_body,
        out_shape=jax.ShapeDtypeStruct((NS, 1, 8), jnp.float32),
    )(px, py, pz, pb)


FOLD = 8
WSLOT = N // FOLD
INFKEY = 0x7F000000
_SORT8 = [(0, 1), (2, 3), (4, 5), (6, 7), (0, 2), (1, 3), (4, 6), (5, 7),
          (1, 2), (5, 6), (0, 4), (1, 5), (2, 6), (3, 7), (2, 4), (3, 5),
          (1, 2), (3, 4), (5, 6)]


BCK = 128


def _knn_body(px_ref, py_ref, pz_ref, cs_ref, nbr_ref):
    px = px_ref[...].reshape(1, FOLD, WSLOT)
    py = py_ref[...].reshape(1, FOLD, WSLOT)
    pz = pz_ref[...].reshape(1, FOLD, WSLOT)
    cs = cs_ref[...]
    cx = cs[:, 0:1].reshape(BCK, 1, 1)
    cy = cs[:, 1:2].reshape(BCK, 1, 1)
    cz = cs[:, 2:3].reshape(BCK, 1, 1)
    dx = cx - px
    dy = cy - py
    dz = cz - pz
    d = dx * dx + dy * dy + dz * dz

    db = lax.bitcast_convert_type(d, jnp.int32)
    sub = lax.broadcasted_iota(jnp.int32, (BCK, FOLD, WSLOT), 1)
    keys = (db & ~(FOLD - 1)) | sub

    r = [keys[:, s, :] for s in range(FOLD)]
    for a, b in _SORT8:
        lo = jnp.minimum(r[a], r[b])
        hi = jnp.maximum(r[a], r[b])
        r[a], r[b] = lo, hi

    ii = lax.broadcasted_iota(jnp.int32, (BCK, WSLOT), 1)
    ik = lax.broadcasted_iota(jnp.int32, (BCK, KTOT), 1)

    def body(k, carry):
        nbr = carry[0]
        r = list(carry[1:])
        m = jnp.min(r[0], axis=1, keepdims=True)
        eq = r[0] == m
        jslot = jnp.min(jnp.where(eq, ii, BIG), axis=1, keepdims=True)
        orig = (m & (FOLD - 1)) * WSLOT + jslot
        nbr = jnp.where(ik == k, orig, nbr)
        hit = ii == jslot
        for s in range(FOLD - 1):
            r[s] = jnp.where(hit, r[s + 1], r[s])
        r[FOLD - 1] = jnp.where(hit, INFKEY, r[FOLD - 1])
        return (nbr, *r)

    out = lax.fori_loop(0, KTOT, body,
                        (jnp.zeros((BCK, KTOT), jnp.int32), *r), unroll=8)
    nbr_ref[...] = out[0]


def _knn(px, py, pz, cs_pad):
    return pl.pallas_call(
        _knn_body,
        grid=(NS_PAD // BCK,),
        in_specs=[
            pl.BlockSpec((8, N // 8), lambda i: (0, 0)),
            pl.BlockSpec((8, N // 8), lambda i: (0, 0)),
            pl.BlockSpec((8, N // 8), lambda i: (0, 0)),
            pl.BlockSpec((BCK, 8), lambda i: (i, 0)),
        ],
        out_specs=pl.BlockSpec((BCK, KTOT), lambda i: (i, 0)),
        out_shape=jax.ShapeDtypeStruct((NS_PAD, KTOT), jnp.int32),
    )(px, py, pz, cs_pad)


def _sc_gather(table, col):
    info = plsc.get_sparse_core_info()
    nw = info.num_cores * info.num_subcores
    rows_per_w = EPAD // nw
    ch = 128
    nch = rows_per_w // ch
    mesh = plsc.VectorSubcoreMesh(core_axis_name="c", subcore_axis_name="s")

    @functools.partial(
        pl.kernel, mesh=mesh,
        out_type=jax.ShapeDtypeStruct((EPAD, TW), jnp.float32),
        scratch_types=[
            pltpu.VMEM((ch,), jnp.int32),
            pltpu.VMEM((ch, TW), jnp.float32),
            pltpu.SemaphoreType.DMA,
        ],
    )
    def gk(table_hbm, col_hbm, out_hbm, idx_v, rows_v, sem):
        wid = lax.axis_index("s") * info.num_cores + lax.axis_index("c")
        base = wid * rows_per_w
        for c in range(nch):
            st = base + c * ch
            pltpu.sync_copy(col_hbm.at[pl.ds(st, ch)], idx_v)
            pltpu.async_copy(table_hbm.at[idx_v], rows_v, sem).wait()
            pltpu.sync_copy(rows_v, out_hbm.at[pl.ds(st, ch)])

    return gk(table, col)


def _table_body(x_ref, p8_ref, w1p0_ref, dwp0_ref, w1p1_ref, dwp1_ref,
                w1x0_ref, dwx0_ref, w1x1_ref, dwx1_ref, t_ref):
    x = x_ref[...]
    p8 = p8_ref[...]

    def mm(a, b):
        return jnp.dot(a, b, preferred_element_type=jnp.float32)

    q0 = mm(x, w1x0_ref[...]) + mm(p8, w1p0_ref[...])
    qd0 = mm(x, dwx0_ref[...]) + mm(p8, dwp0_ref[...])
    q1 = mm(x, w1x1_ref[...]) + mm(p8, w1p1_ref[...])
    qd1 = mm(x, dwx1_ref[...]) + mm(p8, dwp1_ref[...])
    t_ref[...] = jnp.concatenate([q0, qd0, q1, qd1], axis=1)


def _make_table(x, pos8, tws):
    rb = 1000
    full = lambda r, c: pl.BlockSpec((r, c), lambda i: (0, 0))
    in_specs = [pl.BlockSpec((rb, D), lambda i: (i, 0)),
                pl.BlockSpec((rb, 8), lambda i: (i, 0))]
    in_specs += [full(*w.shape) for w in tws]
    return pl.pallas_call(
        _table_body,
        grid=(N // rb,),
        in_specs=in_specs,
        out_specs=pl.BlockSpec((rb, TW), lambda i: (i, 0)),
        out_shape=jax.ShapeDtypeStruct((N, TW), jnp.float32),
    )(x, pos8, *tws)


def _mlpA_body(kk, c1, e_real, o1, od, g_ref, cs_ref, w1p_ref,
               dwp_ref, b1_ref, db_ref, z1_ref, zd_ref, s1_ref, sd_ref):
    i = pl.program_id(0)
    eb = BC * kk
    g = g_ref[...]
    cs = cs_ref[...]

    def rep(a):
        return jnp.broadcast_to(a[:, None, :], (BC, kk, a.shape[1])) \
                  .reshape(eb, a.shape[1])

    def mm(a, b):
        return jnp.dot(a, b, preferred_element_type=jnp.float32)

    z1 = (g[:, :, o1:o1 + c1].reshape(eb, c1)
          - rep(mm(cs, w1p_ref[...])) + b1_ref[...])
    zd = (g[:, :, od:od + D].reshape(eb, D)
          - rep(mm(cs, dwp_ref[...])) + db_ref[...])

    row = lax.broadcasted_iota(jnp.int32, (eb, 1), 0)
    mask = ((row // kk + i * BC) < NS).astype(jnp.float32)

    @pl.when(i == 0)
    def _():
        s1_ref[...] = jnp.zeros_like(s1_ref)
        sd_ref[...] = jnp.zeros_like(sd_ref)

    z1m = z1 * mask
    zdm = zd * mask
    s1_ref[0:1, :] += jnp.sum(z1m, axis=0, keepdims=True)
    s1_ref[1:2, :] += jnp.sum(z1m * z1, axis=0, keepdims=True)
    sd_ref[0:1, :] += jnp.sum(zdm, axis=0, keepdims=True)
    sd_ref[1:2, :] += jnp.sum(zdm * zd, axis=0, keepdims=True)
    z1_ref[...] = z1
    zd_ref[...] = zd


def _bn_coefs(s_ref, e_real, g_ref, bt_ref):
    mu = s_ref[0:1, :] * (1.0 / e_real)
    var = s_ref[1:2, :] * (1.0 / e_real) - mu * mu
    rstd = lax.rsqrt(var + EPS)
    scale = rstd * g_ref[...]
    bias = bt_ref[...] - mu * scale
    return scale, bias


def _mlpB_body(eb, e_real, z_ref, s_ref, g_ref, bt_ref, w_ref, b_ref,
               z2_ref, s2_ref):
    i = pl.program_id(0)
    scale, bias = _bn_coefs(s_ref, e_real, g_ref, bt_ref)
    h = jnp.maximum(z_ref[...] * scale + bias, 0.0)
    z2 = jnp.dot(h, w_ref[...], preferred_element_type=jnp.float32) + b_ref[...]
    row = lax.broadcasted_iota(jnp.int32, (eb, 1), 0)
    mask = ((row + i * eb) < e_real).astype(jnp.float32)

    @pl.when(i == 0)
    def _():
        s2_ref[...] = jnp.zeros_like(s2_ref)

    z2m = z2 * mask
    s2_ref[0:1, :] += jnp.sum(z2m, axis=0, keepdims=True)
    s2_ref[1:2, :] += jnp.sum(z2m * z2, axis=0, keepdims=True)
    z2_ref[...] = z2


def _mlpD_body(kk, e_real, z3_ref, zd_ref, s3_ref, sd_ref, g3_ref, bt3_ref,
               gd_ref, btd_ref, out_ref):
    eb = BC * kk
    sc3, bi3 = _bn_coefs(s3_ref, e_real, g3_ref, bt3_ref)
    scd, bid = _bn_coefs(sd_ref, e_real, gd_ref, btd_ref)
    h = jnp.maximum(z3_ref[...] * sc3 + bi3 + zd_ref[...] * scd + bid, 0.0)
    out_ref[...] = jnp.max(h.reshape(BC, kk, D), axis=1)


def _run_scale(gv, cs_pad, p, kk, o1, od):
    eb = BC * kk
    e_real = NS * kk
    (w1, b1, g1, bt1), (w2, b2, g2, bt2), (w3, b3, g3, bt3) = p["layers"]
    dw, dbl, dg, dbt = p["down"]
    c1, c2, c3 = w1.shape[0], w2.shape[0], w3.shape[0]

    w1pT = jnp.zeros((8, c1), jnp.float32).at[:3].set(w1[:, D:].T)
    dwpT = jnp.zeros((8, D), jnp.float32).at[:3].set(dw[:, D:].T)

    def row(v):
        return v.reshape(1, -1)

    full = lambda r, c: pl.BlockSpec((r, c), lambda i: (0, 0))
    z1, zd, s1, sd = pl.pallas_call(
        functools.partial(_mlpA_body, kk, c1, e_real, o1, od),
        grid=(GRID_C,),
        in_specs=[
            pl.BlockSpec((BC, kk, TW), lambda i: (i, 0, 0)),
            pl.BlockSpec((BC, 8), lambda i: (i, 0)),
            full(8, c1), full(8, D),
            full(1, c1), full(1, D),
        ],
        out_specs=[
            pl.BlockSpec((eb, c1), lambda i: (i, 0)),
            pl.BlockSpec((eb, D), lambda i: (i, 0)),
            full(8, c1), full(8, D),
        ],
        out_shape=[
            jax.ShapeDtypeStruct((GRID_C * eb, c1), jnp.float32),
            jax.ShapeDtypeStruct((GRID_C * eb, D), jnp.float32),
            jax.ShapeDtypeStruct((8, c1), jnp.float32),
            jax.ShapeDtypeStruct((8, D), jnp.float32),
        ],
    )(gv, cs_pad, w1pT, dwpT, row(b1), row(dbl))

    def bc_layer(z, s, g_, bt_, w_, b_, cin, cout):
        return pl.pallas_call(
            functools.partial(_mlpB_body, eb, e_real),
            grid=(GRID_C,),
            in_specs=[
                pl.BlockSpec((eb, cin), lambda i: (i, 0)),
                full(8, cin), full(1, cin), full(1, cin),
                full(cin, cout), full(1, cout),
            ],
            out_specs=[
                pl.BlockSpec((eb, cout), lambda i: (i, 0)),
                full(8, cout),
            ],
            out_shape=[
                jax.ShapeDtypeStruct((GRID_C * eb, cout), jnp.float32),
                jax.ShapeDtypeStruct((8, cout), jnp.float32),
            ],
        )(z, s, row(g_), row(bt_), w_.T, row(b_))

    z2, s2 = bc_layer(z1, s1, g1, bt1, w2, b2, c1, c2)
    z3, s3 = bc_layer(z2, s2, g2, bt2, w3, b3, c2, c3)

    out = pl.pallas_call(
        functools.partial(_mlpD_body, kk, e_real),
        grid=(GRID_C,),
        in_specs=[
            pl.BlockSpec((eb, D), lambda i: (i, 0)),
            pl.BlockSpec((eb, D), lambda i: (i, 0)),
            full(8, D), full(8, D),
            full(1, D), full(1, D), full(1, D), full(1, D),
        ],
        out_specs=pl.BlockSpec((BC, D), lambda i: (i, 0)),
        out_shape=jax.ShapeDtypeStruct((NS_PAD, D), jnp.float32),
    )(z3, zd, s3, sd, row(g3), row(bt3), row(dg), row(dbt))
    return out[:NS]


def kernel(x, pos, batch, params):
    pos = pos.astype(jnp.float32)
    x = x.astype(jnp.float32)
    px = pos[:, 0].reshape(8, N // 8)
    py = pos[:, 1].reshape(8, N // 8)
    pz = pos[:, 2].reshape(8, N // 8)
    pb = batch.astype(jnp.float32).reshape(8, N // 8)

    rows = _fps(px, py, pz, pb).reshape(NS, 8)
    pos_s = rows[:, :3]
    batch_s = rows[:, 3].astype(jnp.int32)

    cs_pad = jnp.zeros((NS_PAD, 8), jnp.float32).at[:NS].set(rows)
    nbr = _knn(px, py, pz, cs_pad)

    pos8 = jnp.zeros((N, 8), jnp.float32).at[:, :3].set(pos)

    def wsplit(w, cout):
        wxT = w[:, :D].T
        wpT = jnp.zeros((8, cout), jnp.float32).at[:3].set(w[:, D:].T)
        return wxT, wpT

    w10, b0 = params[0]["layers"][0][0], params[0]["down"][0]
    w11, b1w = params[1]["layers"][0][0], params[1]["down"][0]
    w1x0, w1p0 = wsplit(w10, 64)
    dwx0, dwp0 = wsplit(b0, D)
    w1x1, w1p1 = wsplit(w11, 64)
    dwx1, dwp1 = wsplit(b1w, D)
    table = _make_table(
        x, pos8, [w1p0, dwp0, w1p1, dwp1, w1x0, dwx0, w1x1, dwx1])

    g = _sc_gather(table, nbr.reshape(-1))
    gv = g.reshape(NS_PAD, KTOT, TW)

    out0 = _run_scale(gv, cs_pad, params[0], 16, 0, 64)
    out1 = _run_scale(gv, cs_pad, params[1], 32, 192, 256)
    return jnp.concatenate([out0, out1], axis=1), pos_s, batch_s


# speedup vs baseline: 1.7362x; 1.0715x over previous
import functools

import jax
import jax.numpy as jnp
from jax import lax
from jax.experimental import pallas as pl
from jax.experimental.pallas import tpu as pltpu
from jax.experimental.pallas import tpu_sc as plsc

N = 10000
NS = 2500
NS_PAD = 2560
BC = 128
GRID_C = NS_PAD // BC
KTOT = 32
EPAD = NS_PAD * KTOT
D = 128
TW = 384
EPS = 1e-5
BIG = 2 ** 30


def _fps_body(px_ref, py_ref, pz_ref, pb_ref, rows_ref):
    w = N // 8
    pxv = px_ref[...]
    pyv = py_ref[...]
    pzv = pz_ref[...]
    pbv = pb_ref[...]
    ii = (lax.broadcasted_iota(jnp.int32, (8, w), 0) * w
          + lax.broadcasted_iota(jnp.int32, (8, w), 1))
    lane8 = lax.broadcasted_iota(jnp.int32, (1, 1, 8), 2)

    def pick(i, j):
        m = ii == j
        cx = jnp.sum(jnp.where(m, pxv, 0.0))
        cy = jnp.sum(jnp.where(m, pyv, 0.0))
        cz = jnp.sum(jnp.where(m, pzv, 0.0))
        cb = jnp.sum(jnp.where(m, pbv, 0.0))
        vals = jnp.where(
            lane8 == 0, cx,
            jnp.where(lane8 == 1, cy,
                      jnp.where(lane8 == 2, cz,
                                jnp.where(lane8 == 3, cb, 0.0))))
        rows_ref[pl.ds(i, 1)] = vals
        return cx, cy, cz

    cx0, cy0, cz0 = pick(0, 0)
    dists0 = jnp.full((8, w), jnp.inf, dtype=jnp.float32)

    def body(i, carry):
        cx, cy, cz, dists = carry
        d = (pxv - cx) * (pxv - cx) + (pyv - cy) * (pyv - cy) \
            + (pzv - cz) * (pzv - cz)
        dists = jnp.minimum(dists, d)
        m = jnp.max(dists)
        j = jnp.min(jnp.where(dists == m, ii, BIG))
        ncx, ncy, ncz = pick(i, j)
        return ncx, ncy, ncz, dists

    lax.fori_loop(1, NS, body, (cx0, cy0, cz0, dists0), unroll=8)


def _fps(px, py, pz, pb):
    return pl.pallas_call(
        _fps_body,
        out_shape=jax.ShapeDtypeStruct((NS, 1, 8), jnp.float32),
    )(px, py, pz, pb)


FOLD = 16
STACK = 5
WSLOT = N // FOLD
INFKEY = 0x7F000000
BCK = 128


def _batcher_pairs(n):
    pairs = []
    p = 1
    while p < n:
        k = p
        while k >= 1:
            for j in range(k % p, n - k, 2 * k):
                for i in range(0, min(k, n - j - k)):
                    if (i + j) // (2 * p) == (i + j + k) // (2 * p):
                        pairs.append((i + j, i + j + k))
            k //= 2
        p *= 2
    return pairs


def _knn_body(px_ref, py_ref, pz_ref, cs_ref, nbr_ref):
    px = px_ref[...].reshape(1, FOLD, WSLOT)
    py = py_ref[...].reshape(1, FOLD, WSLOT)
    pz = pz_ref[...].reshape(1, FOLD, WSLOT)
    cs = cs_ref[...]
    cx = cs[:, 0:1].reshape(BCK, 1, 1)
    cy = cs[:, 1:2].reshape(BCK, 1, 1)
    cz = cs[:, 2:3].reshape(BCK, 1, 1)
    dx = cx - px
    dy = cy - py
    dz = cz - pz
    d = dx * dx + dy * dy + dz * dz

    db = lax.bitcast_convert_type(d, jnp.int32)
    sub = lax.broadcasted_iota(jnp.int32, (BCK, FOLD, WSLOT), 1)
    keys = (db & ~(FOLD - 1)) | sub

    r = [keys[:, s, :] for s in range(FOLD)]
    for a, b in _batcher_pairs(FOLD):
        lo = jnp.minimum(r[a], r[b])
        hi = jnp.maximum(r[a], r[b])
        r[a], r[b] = lo, hi
    r = r[:STACK]

    ii = lax.broadcasted_iota(jnp.int32, (BCK, WSLOT), 1)
    ik = lax.broadcasted_iota(jnp.int32, (BCK, KTOT), 1)

    def body(k, carry):
        nbr = carry[0]
        r = list(carry[1:])
        m = jnp.min(r[0], axis=1, keepdims=True)
        eq = r[0] == m
        jslot = jnp.min(jnp.where(eq, ii, BIG), axis=1, keepdims=True)
        orig = (m & (FOLD - 1)) * WSLOT + jslot
        nbr = jnp.where(ik == k, orig, nbr)
        hit = ii == jslot
        for s in range(STACK - 1):
            r[s] = jnp.where(hit, r[s + 1], r[s])
        r[STACK - 1] = jnp.where(hit, INFKEY, r[STACK - 1])
        return (nbr, *r)

    out = lax.fori_loop(0, KTOT, body,
                        (jnp.zeros((BCK, KTOT), jnp.int32), *r), unroll=8)
    nbr_ref[...] = out[0]


def _knn(px, py, pz, cs_pad):
    return pl.pallas_call(
        _knn_body,
        grid=(NS_PAD // BCK,),
        in_specs=[
            pl.BlockSpec((FOLD, WSLOT), lambda i: (0, 0)),
            pl.BlockSpec((FOLD, WSLOT), lambda i: (0, 0)),
            pl.BlockSpec((FOLD, WSLOT), lambda i: (0, 0)),
            pl.BlockSpec((BCK, 8), lambda i: (i, 0)),
        ],
        out_specs=pl.BlockSpec((BCK, KTOT), lambda i: (i, 0)),
        out_shape=jax.ShapeDtypeStruct((NS_PAD, KTOT), jnp.int32),
    )(px, py, pz, cs_pad)


def _sc_gather(table, col):
    info = plsc.get_sparse_core_info()
    nw = info.num_cores * info.num_subcores
    rows_per_w = EPAD // nw
    ch = 128
    nch = rows_per_w // ch
    mesh = plsc.VectorSubcoreMesh(core_axis_name="c", subcore_axis_name="s")

    @functools.partial(
        pl.kernel, mesh=mesh,
        out_type=jax.ShapeDtypeStruct((EPAD, TW), jnp.float32),
        scratch_types=[
            pltpu.VMEM((ch,), jnp.int32),
            pltpu.VMEM((ch, TW), jnp.float32),
            pltpu.SemaphoreType.DMA,
        ],
    )
    def gk(table_hbm, col_hbm, out_hbm, idx_v, rows_v, sem):
        wid = lax.axis_index("s") * info.num_cores + lax.axis_index("c")
        base = wid * rows_per_w
        for c in range(nch):
            st = base + c * ch
            pltpu.sync_copy(col_hbm.at[pl.ds(st, ch)], idx_v)
            pltpu.async_copy(table_hbm.at[idx_v], rows_v, sem).wait()
            pltpu.sync_copy(rows_v, out_hbm.at[pl.ds(st, ch)])

    return gk(table, col)


def _table_body(x_ref, p8_ref, w1p0_ref, dwp0_ref, w1p1_ref, dwp1_ref,
                w1x0_ref, dwx0_ref, w1x1_ref, dwx1_ref, t_ref):
    x = x_ref[...]
    p8 = p8_ref[...]

    def mm(a, b):
        return jnp.dot(a, b, preferred_element_type=jnp.float32)

    q0 = mm(x, w1x0_ref[...]) + mm(p8, w1p0_ref[...])
    qd0 = mm(x, dwx0_ref[...]) + mm(p8, dwp0_ref[...])
    q1 = mm(x, w1x1_ref[...]) + mm(p8, w1p1_ref[...])
    qd1 = mm(x, dwx1_ref[...]) + mm(p8, dwp1_ref[...])
    t_ref[...] = jnp.concatenate([q0, qd0, q1, qd1], axis=1)


def _make_table(x, pos8, tws):
    rb = 1000
    full = lambda r, c: pl.BlockSpec((r, c), lambda i: (0, 0))
    in_specs = [pl.BlockSpec((rb, D), lambda i: (i, 0)),
                pl.BlockSpec((rb, 8), lambda i: (i, 0))]
    in_specs += [full(*w.shape) for w in tws]
    return pl.pallas_call(
        _table_body,
        grid=(N // rb,),
        in_specs=in_specs,
        out_specs=pl.BlockSpec((rb, TW), lambda i: (i, 0)),
        out_shape=jax.ShapeDtypeStruct((N, TW), jnp.float32),
    )(x, pos8, *tws)


def _mlpA_body(kk, c1, e_real, o1, od, g_ref, cs_ref, w1p_ref,
               dwp_ref, b1_ref, db_ref, z1_ref, zd_ref, s1_ref, sd_ref):
    i = pl.program_id(0)
    eb = BC * kk
    g = g_ref[...]
    cs = cs_ref[...]

    def rep(a):
        return jnp.broadcast_to(a[:, None, :], (BC, kk, a.shape[1])) \
                  .reshape(eb, a.shape[1])

    def mm(a, b):
        return jnp.dot(a, b, preferred_element_type=jnp.float32)

    z1 = (g[:, :, o1:o1 + c1].reshape(eb, c1)
          - rep(mm(cs, w1p_ref[...])) + b1_ref[...])
    zd = (g[:, :, od:od + D].reshape(eb, D)
          - rep(mm(cs, dwp_ref[...])) + db_ref[...])

    row = lax.broadcasted_iota(jnp.int32, (eb, 1), 0)
    mask = ((row // kk + i * BC) < NS).astype(jnp.float32)

    @pl.when(i == 0)
    def _():
        s1_ref[...] = jnp.zeros_like(s1_ref)
        sd_ref[...] = jnp.zeros_like(sd_ref)

    z1m = z1 * mask
    zdm = zd * mask
    s1_ref[0:1, :] += jnp.sum(z1m, axis=0, keepdims=True)
    s1_ref[1:2, :] += jnp.sum(z1m * z1, axis=0, keepdims=True)
    sd_ref[0:1, :] += jnp.sum(zdm, axis=0, keepdims=True)
    sd_ref[1:2, :] += jnp.sum(zdm * zd, axis=0, keepdims=True)
    z1_ref[...] = z1
    zd_ref[...] = zd


def _bn_coefs(s_ref, e_real, g_ref, bt_ref):
    mu = s_ref[0:1, :] * (1.0 / e_real)
    var = s_ref[1:2, :] * (1.0 / e_real) - mu * mu
    rstd = lax.rsqrt(var + EPS)
    scale = rstd * g_ref[...]
    bias = bt_ref[...] - mu * scale
    return scale, bias


def _mlpB_body(eb, e_real, z_ref, s_ref, g_ref, bt_ref, w_ref, b_ref,
               z2_ref, s2_ref):
    i = pl.program_id(0)
    scale, bias = _bn_coefs(s_ref, e_real, g_ref, bt_ref)
    h = jnp.maximum(z_ref[...] * scale + bias, 0.0)
    z2 = jnp.dot(h, w_ref[...], preferred_element_type=jnp.float32) + b_ref[...]
    row = lax.broadcasted_iota(jnp.int32, (eb, 1), 0)
    mask = ((row + i * eb) < e_real).astype(jnp.float32)

    @pl.when(i == 0)
    def _():
        s2_ref[...] = jnp.zeros_like(s2_ref)

    z2m = z2 * mask
    s2_ref[0:1, :] += jnp.sum(z2m, axis=0, keepdims=True)
    s2_ref[1:2, :] += jnp.sum(z2m * z2, axis=0, keepdims=True)
    z2_ref[...] = z2


def _mlpD_body(kk, e_real, z3_ref, zd_ref, s3_ref, sd_ref, g3_ref, bt3_ref,
               gd_ref, btd_ref, out_ref):
    eb = BC * kk
    sc3, bi3 = _bn_coefs(s3_ref, e_real, g3_ref, bt3_ref)
    scd, bid = _bn_coefs(sd_ref, e_real, gd_ref, btd_ref)
    h = jnp.maximum(z3_ref[...] * sc3 + bi3 + zd_ref[...] * scd + bid, 0.0)
    out_ref[...] = jnp.max(h.reshape(BC, kk, D), axis=1)


def _run_scale(gv, cs_pad, p, kk, o1, od):
    eb = BC * kk
    e_real = NS * kk
    (w1, b1, g1, bt1), (w2, b2, g2, bt2), (w3, b3, g3, bt3) = p["layers"]
    dw, dbl, dg, dbt = p["down"]
    c1, c2, c3 = w1.shape[0], w2.shape[0], w3.shape[0]

    w1pT = jnp.zeros((8, c1), jnp.float32).at[:3].set(w1[:, D:].T)
    dwpT = jnp.zeros((8, D), jnp.float32).at[:3].set(dw[:, D:].T)

    def row(v):
        return v.reshape(1, -1)

    full = lambda r, c: pl.BlockSpec((r, c), lambda i: (0, 0))
    z1, zd, s1, sd = pl.pallas_call(
        functools.partial(_mlpA_body, kk, c1, e_real, o1, od),
        grid=(GRID_C,),
        in_specs=[
            pl.BlockSpec((BC, kk, TW), lambda i: (i, 0, 0)),
            pl.BlockSpec((BC, 8), lambda i: (i, 0)),
            full(8, c1), full(8, D),
            full(1, c1), full(1, D),
        ],
        out_specs=[
            pl.BlockSpec((eb, c1), lambda i: (i, 0)),
            pl.BlockSpec((eb, D), lambda i: (i, 0)),
            full(8, c1), full(8, D),
        ],
        out_shape=[
            jax.ShapeDtypeStruct((GRID_C * eb, c1), jnp.float32),
            jax.ShapeDtypeStruct((GRID_C * eb, D), jnp.float32),
            jax.ShapeDtypeStruct((8, c1), jnp.float32),
            jax.ShapeDtypeStruct((8, D), jnp.float32),
        ],
    )(gv, cs_pad, w1pT, dwpT, row(b1), row(dbl))

    def bc_layer(z, s, g_, bt_, w_, b_, cin, cout):
        return pl.pallas_call(
            functools.partial(_mlpB_body, eb, e_real),
            grid=(GRID_C,),
            in_specs=[
                pl.BlockSpec((eb, cin), lambda i: (i, 0)),
                full(8, cin), full(1, cin), full(1, cin),
                full(cin, cout), full(1, cout),
            ],
            out_specs=[
                pl.BlockSpec((eb, cout), lambda i: (i, 0)),
                full(8, cout),
            ],
            out_shape=[
                jax.ShapeDtypeStruct((GRID_C * eb, cout), jnp.float32),
                jax.ShapeDtypeStruct((8, cout), jnp.float32),
            ],
        )(z, s, row(g_), row(bt_), w_.T, row(b_))

    z2, s2 = bc_layer(z1, s1, g1, bt1, w2, b2, c1, c2)
    z3, s3 = bc_layer(z2, s2, g2, bt2, w3, b3, c2, c3)

    out = pl.pallas_call(
        functools.partial(_mlpD_body, kk, e_real),
        grid=(GRID_C,),
        in_specs=[
            pl.BlockSpec((eb, D), lambda i: (i, 0)),
            pl.BlockSpec((eb, D), lambda i: (i, 0)),
            full(8, D), full(8, D),
            full(1, D), full(1, D), full(1, D), full(1, D),
        ],
        out_specs=pl.BlockSpec((BC, D), lambda i: (i, 0)),
        out_shape=jax.ShapeDtypeStruct((NS_PAD, D), jnp.float32),
    )(z3, zd, s3, sd, row(g3), row(bt3), row(dg), row(dbt))
    return out[:NS]


def kernel(x, pos, batch, params):
    pos = pos.astype(jnp.float32)
    x = x.astype(jnp.float32)
    px = pos[:, 0].reshape(8, N // 8)
    py = pos[:, 1].reshape(8, N // 8)
    pz = pos[:, 2].reshape(8, N // 8)
    pb = batch.astype(jnp.float32).reshape(8, N // 8)

    rows = _fps(px, py, pz, pb).reshape(NS, 8)
    pos_s = rows[:, :3]
    batch_s = rows[:, 3].astype(jnp.int32)

    cs_pad = jnp.zeros((NS_PAD, 8), jnp.float32).at[:NS].set(rows)
    nbr = _knn(pos[:, 0].reshape(FOLD, WSLOT),
               pos[:, 1].reshape(FOLD, WSLOT),
               pos[:, 2].reshape(FOLD, WSLOT), cs_pad)

    pos8 = jnp.zeros((N, 8), jnp.float32).at[:, :3].set(pos)

    def wsplit(w, cout):
        wxT = w[:, :D].T
        wpT = jnp.zeros((8, cout), jnp.float32).at[:3].set(w[:, D:].T)
        return wxT, wpT

    w10, b0 = params[0]["layers"][0][0], params[0]["down"][0]
    w11, b1w = params[1]["layers"][0][0], params[1]["down"][0]
    w1x0, w1p0 = wsplit(w10, 64)
    dwx0, dwp0 = wsplit(b0, D)
    w1x1, w1p1 = wsplit(w11, 64)
    dwx1, dwp1 = wsplit(b1w, D)
    table = _make_table(
        x, pos8, [w1p0, dwp0, w1p1, dwp1, w1x0, dwx0, w1x1, dwx1])

    g = _sc_gather(table, nbr.reshape(-1))
    gv = g.reshape(NS_PAD, KTOT, TW)

    out0 = _run_scale(gv, cs_pad, params[0], 16, 0, 64)
    out1 = _run_scale(gv, cs_pad, params[1], 32, 192, 256)
    return jnp.concatenate([out0, out1], axis=1), pos_s, batch_s
